# Initial kernel scaffold; baseline (speedup 1.0000x reference)
import jax
import jax.numpy as jnp
from jax.experimental import pallas as pl


def kernel(n_feats, edge_index, edge_attr, W1, attn_l1, attn_r1, b1, W2, attn_l2, attn_r2, b2, Wp1, bp1, gamma, beta, Wp2, bp2):
    raise NotImplementedError("write your pallas kernel here")


# speedup vs baseline: 1.1512x; 1.1512x over previous
import functools

import jax
import jax.numpy as jnp
from jax.experimental import pallas as pl


def _mm_body(x_ref, w_ref, o_ref):
    o_ref[...] = jnp.dot(x_ref[...], w_ref[...],
                         preferred_element_type=jnp.float32)


def _matmul(x, w, block_rows=2000):
    n, k = x.shape
    m = w.shape[1]
    return pl.pallas_call(
        _mm_body,
        out_shape=jax.ShapeDtypeStruct((n, m), jnp.float32),
        grid=(n // block_rows,),
        in_specs=[pl.BlockSpec((block_rows, k), lambda i: (i, 0)),
                  pl.BlockSpec((k, m), lambda i: (0, 0))],
        out_specs=pl.BlockSpec((block_rows, m), lambda i: (i, 0)),
    )(x, w)


def _gat_layer(x, src, dst, W, al, ar, b, H, O):
    n = x.shape[0]
    h = _matmul(x, W.T).reshape(n, H, O)
    el = jnp.sum(h * al, axis=-1)
    er = jnp.sum(h * ar, axis=-1)
    e = el[src] + er[dst]
    e = jnp.where(e > 0, e, 0.2 * e)
    ex = jnp.exp(e)
    es = jax.ops.segment_sum(ex, dst, num_segments=n)
    agg = jax.ops.segment_sum(h[src] * ex[:, :, None], dst, num_segments=n)
    out = agg / (es[:, :, None] + 1e-9) + b.reshape(1, H, O)
    return jax.nn.relu(out).reshape(n, H * O)


def kernel(n_feats, edge_index, edge_attr, W1, attn_l1, attn_r1, b1,
           W2, attn_l2, attn_r2, b2, Wp1, bp1, gamma, beta, Wp2, bp2):
    src = edge_index[0]
    dst = edge_index[1]
    d_edge = edge_attr.shape[1]
    h1 = _gat_layer(n_feats, src, dst, W1, attn_l1, attn_r1, b1, 8, 16)
    h2 = _gat_layer(h1, src, dst, W2, attn_l2, attn_r2, b2, 1, 32)
    o2 = h2.shape[1]
    wa = Wp1[:, :d_edge]
    wsrc = Wp1[:, d_edge:d_edge + o2]
    wdst = Wp1[:, d_edge + o2:]
    zs = h2 @ wsrc.T
    zd = h2 @ wdst.T
    z = edge_attr @ wa.T + zs[src] + zd[dst] + bp1
    mean = jnp.mean(z, axis=0)
    var = jnp.var(z, axis=0)
    z = (z - mean) / jnp.sqrt(var + 1e-5) * gamma + beta
    logits = z @ Wp2.T + bp2
    return jax.nn.log_softmax(logits, axis=1)


# speedup vs baseline: 6.8815x; 5.9775x over previous
import jax
import jax.numpy as jnp
from jax import lax
from jax.experimental import pallas as pl
from jax.experimental.pallas import tpu as pltpu
from jax.experimental.pallas import tpu_sc as plsc

N = 10000
E = 320000
NC, NS, L = 2, 16, 16
NW = NC * NS
EPW = E // NW
C = 80
NCHUNKS = EPW // C
NPAD = 10240

_SC_PARAMS = pltpu.CompilerParams(use_tc_tiling_on_sc=False,
                                  needs_layout_passes=False)
_MESH = plsc.VectorSubcoreMesh(core_axis_name="c", subcore_axis_name="s",
                               num_cores=NC, num_subcores=NS)


def _mm_body(x_ref, w_ref, o_ref):
    o_ref[...] = jnp.dot(x_ref[...], w_ref[...],
                         preferred_element_type=jnp.float32)


def _matmul(x, w, block_rows=2000):
    n, k = x.shape
    m = w.shape[1]
    return pl.pallas_call(
        _mm_body,
        out_shape=jax.ShapeDtypeStruct((n, m), jnp.float32),
        grid=(n // block_rows,),
        in_specs=[pl.BlockSpec((block_rows, k), lambda i: (i, 0)),
                  pl.BlockSpec((k, m), lambda i: (0, 0))],
        out_specs=pl.BlockSpec((block_rows, m), lambda i: (i, 0)),
    )(x, w)


def _sc1_body(ht_h, elt_h, ert_h, src_h, dst_h, out_h,
              acc, sidx, didx, rows, elr, err, msg, exb):
    cid = lax.axis_index("c")
    sid = lax.axis_index("s")
    wid = cid * NS + sid
    zv = jnp.zeros((L,), jnp.float32)
    iota = lax.iota(jnp.int32, L)

    @plsc.parallel_loop(0, C, 1, unroll=4)
    def _zrow(r):
        for j in range(9):
            msg[r, pl.ds(j * L, L)] = zv
    for k in range(NPAD // NS // C):
        pltpu.sync_copy(msg, acc.at[pl.ds((sid * 8 + k) * C, C)])
    plsc.subcore_barrier()

    ebase = wid * EPW

    def chunk(c, _):
        base = ebase + c * C
        pltpu.sync_copy(src_h.at[pl.ds(base, C)], sidx)
        pltpu.sync_copy(dst_h.at[pl.ds(base, C)], didx)
        pltpu.sync_copy(ht_h.at[sidx], rows)
        pltpu.sync_copy(elt_h.at[sidx], elr)
        pltpu.sync_copy(ert_h.at[didx], err)

        def grp(g, _):
            rid = g * L + iota
            for h in range(8):
                hf = jnp.full((L,), h, jnp.int32)
                e = (plsc.load_gather(elr, [rid, hf])
                     + plsc.load_gather(err, [rid, hf]))
                e = jnp.maximum(e, 0.2 * e)
                exb[h, pl.ds(g * L, L)] = jnp.exp(e)
            return 0
        lax.fori_loop(0, C // L, grp, 0)

        @plsc.parallel_loop(0, C, 1, unroll=2)
        def _edge(e):
            ef = jnp.full((L,), e, jnp.int32)
            wv = plsc.load_gather(exb, [iota % 8, ef])
            msg[e, pl.ds(128, L)] = wv
            for j in range(8):
                w = plsc.load_gather(exb, [jnp.full((L,), j, jnp.int32), ef])
                msg[e, pl.ds(j * L, L)] = w * rows[e, pl.ds(j * L, L)]

        pltpu.sync_copy(msg, acc.at[didx], add=True)
        return 0
    lax.fori_loop(0, NCHUNKS, chunk, 0)

    plsc.subcore_barrier()
    rpw = NPAD // NS
    pltpu.sync_copy(acc.at[pl.ds(sid * rpw, rpw)],
                    out_h.at[cid, pl.ds(sid * rpw, rpw)])


_sc1 = pl.kernel(
    _sc1_body,
    out_type=jax.ShapeDtypeStruct((NC, NPAD, 144), jnp.float32),
    mesh=_MESH,
    compiler_params=_SC_PARAMS,
    scratch_types=[
        pltpu.VMEM_SHARED((NPAD, 144), jnp.float32),
        pltpu.VMEM((C,), jnp.int32),
        pltpu.VMEM((C,), jnp.int32),
        pltpu.VMEM((C, 128), jnp.float32),
        pltpu.VMEM((C, 16), jnp.float32),
        pltpu.VMEM((C, 16), jnp.float32),
        pltpu.VMEM((C, 144), jnp.float32),
        pltpu.VMEM((8, C), jnp.float32),
    ],
)


def _gat_layer1(x, src, dst, W, al, ar, b):
    n = x.shape[0]
    h = _matmul(x, W.T)
    alT = (jnp.eye(8, dtype=jnp.float32)[:, None, :]
           * al[:, :, None]).reshape(128, 8)
    arT = (jnp.eye(8, dtype=jnp.float32)[:, None, :]
           * ar[:, :, None]).reshape(128, 8)
    el = h @ alT
    er = h @ arT
    z8 = jnp.zeros((n, 8), jnp.float32)
    elt = jnp.concatenate([el, z8], axis=1)
    ert = jnp.concatenate([er, z8], axis=1)
    out = _sc1(h, elt, ert, src, dst)
    agg = (out[0, :n, :128] + out[1, :n, :128]).reshape(n, 8, 16)
    es = out[0, :n, 128:136] + out[1, :n, 128:136]
    o = agg / (es[:, :, None] + 1e-9) + b.reshape(1, 8, 16)
    return jax.nn.relu(o).reshape(n, 128)


def _gat_layer2_jax(x, src, dst, W, al, ar, b, H, O):
    n = x.shape[0]
    h = _matmul(x, W.T).reshape(n, H, O)
    el = jnp.sum(h * al, axis=-1)
    er = jnp.sum(h * ar, axis=-1)
    e = el[src] + er[dst]
    e = jnp.where(e > 0, e, 0.2 * e)
    ex = jnp.exp(e)
    es = jax.ops.segment_sum(ex, dst, num_segments=n)
    agg = jax.ops.segment_sum(h[src] * ex[:, :, None], dst, num_segments=n)
    out = agg / (es[:, :, None] + 1e-9) + b.reshape(1, H, O)
    return jax.nn.relu(out).reshape(n, H * O)


def kernel(n_feats, edge_index, edge_attr, W1, attn_l1, attn_r1, b1,
           W2, attn_l2, attn_r2, b2, Wp1, bp1, gamma, beta, Wp2, bp2):
    src = edge_index[0]
    dst = edge_index[1]
    d_edge = edge_attr.shape[1]
    h1 = _gat_layer1(n_feats, src, dst, W1, attn_l1, attn_r1, b1)
    h2 = _gat_layer2_jax(h1, src, dst, W2, attn_l2, attn_r2, b2, 1, 32)
    o2 = h2.shape[1]
    wa = Wp1[:, :d_edge]
    wsrc = Wp1[:, d_edge:d_edge + o2]
    wdst = Wp1[:, d_edge + o2:]
    zs = h2 @ wsrc.T
    zd = h2 @ wdst.T
    z = edge_attr @ wa.T + zs[src] + zd[dst] + bp1
    mean = jnp.mean(z, axis=0)
    var = jnp.var(z, axis=0)
    z = (z - mean) / jnp.sqrt(var + 1e-5) * gamma + beta
    logits = z @ Wp2.T + bp2
    return jax.nn.log_softmax(logits, axis=1)


# speedup vs baseline: 25.0544x; 3.6408x over previous
import jax
import jax.numpy as jnp
from jax import lax
from jax.experimental import pallas as pl
from jax.experimental.pallas import tpu as pltpu
from jax.experimental.pallas import tpu_sc as plsc

N = 10000
E = 320000
NC, NS, L = 2, 16, 16
NW = NC * NS
EPW = E // NW
C = 80
NCHUNKS = EPW // C
NPAD = 10240
BN = 2000
BE = 4000
NBE = E // BE

_SC_PARAMS = pltpu.CompilerParams(use_tc_tiling_on_sc=False,
                                  needs_layout_passes=False)
_MESH = plsc.VectorSubcoreMesh(core_axis_name="c", subcore_axis_name="s",
                               num_cores=NC, num_subcores=NS)


def _tc1_body(x_ref, w_ref, alT_ref, arT_ref, h_ref, elt_ref, ert_ref):
    h = jnp.dot(x_ref[...], w_ref[...], preferred_element_type=jnp.float32)
    h_ref[...] = h
    el = jnp.dot(h, alT_ref[...], preferred_element_type=jnp.float32)
    er = jnp.dot(h, arT_ref[...], preferred_element_type=jnp.float32)
    z = jnp.zeros_like(el)
    elt_ref[...] = jnp.concatenate([el, z], axis=1)
    ert_ref[...] = jnp.concatenate([er, z], axis=1)


def _tc1(x, w1t, alT, arT):
    return pl.pallas_call(
        _tc1_body,
        out_shape=[jax.ShapeDtypeStruct((N, 128), jnp.float32),
                   jax.ShapeDtypeStruct((N, 16), jnp.float32),
                   jax.ShapeDtypeStruct((N, 16), jnp.float32)],
        grid=(N // BN,),
        in_specs=[pl.BlockSpec((BN, 128), lambda i: (i, 0)),
                  pl.BlockSpec((128, 128), lambda i: (0, 0)),
                  pl.BlockSpec((128, 8), lambda i: (0, 0)),
                  pl.BlockSpec((128, 8), lambda i: (0, 0))],
        out_specs=[pl.BlockSpec((BN, 128), lambda i: (i, 0)),
                   pl.BlockSpec((BN, 16), lambda i: (i, 0)),
                   pl.BlockSpec((BN, 16), lambda i: (i, 0))],
    )(x, w1t, alT, arT)


def _tc2_body(o_ref, b1_ref, ex8_ref, w2t_ref, a2t_ref,
              ht2_ref, elt2_ref, ert2_ref):
    agg = o_ref[0, :, :128] + o_ref[1, :, :128]
    es = o_ref[0, :, 128:136] + o_ref[1, :, 128:136]
    es128 = jnp.dot(es, ex8_ref[...], preferred_element_type=jnp.float32)
    h1 = jnp.maximum(agg / (es128 + 1e-9) + b1_ref[...], 0.0)
    h2p = jnp.dot(h1, w2t_ref[...], preferred_element_type=jnp.float32)
    ea2 = jnp.dot(h2p, a2t_ref[...], preferred_element_type=jnp.float32)
    ht2_ref[...] = h2p
    z15 = jnp.zeros((h2p.shape[0], 15), jnp.float32)
    elt2_ref[...] = jnp.concatenate([ea2[:, 0:1], z15], axis=1)
    ert2_ref[...] = jnp.concatenate([ea2[:, 1:2], z15], axis=1)


def _tc2(out1, b1, ex8, w2t, a2t):
    return pl.pallas_call(
        _tc2_body,
        out_shape=[jax.ShapeDtypeStruct((N, 32), jnp.float32),
                   jax.ShapeDtypeStruct((N, 16), jnp.float32),
                   jax.ShapeDtypeStruct((N, 16), jnp.float32)],
        grid=(N // BN,),
        in_specs=[pl.BlockSpec((NC, BN, 144), lambda i: (0, i, 0)),
                  pl.BlockSpec((1, 128), lambda i: (0, 0)),
                  pl.BlockSpec((8, 128), lambda i: (0, 0)),
                  pl.BlockSpec((128, 32), lambda i: (0, 0)),
                  pl.BlockSpec((32, 2), lambda i: (0, 0))],
        out_specs=[pl.BlockSpec((BN, 32), lambda i: (i, 0)),
                   pl.BlockSpec((BN, 16), lambda i: (i, 0)),
                   pl.BlockSpec((BN, 16), lambda i: (i, 0))],
    )(out1, b1, ex8, w2t, a2t)


def _tc3_body(o_ref, b2_ref, wsdT_ref, zsd_ref, zdd_ref):
    agg = o_ref[0, :, :32] + o_ref[1, :, :32]
    es = o_ref[0, :, 32:33] + o_ref[1, :, 32:33]
    h2 = jnp.maximum(agg / (es + 1e-9) + b2_ref[...], 0.0)
    zz = jnp.dot(h2, wsdT_ref[...], preferred_element_type=jnp.float32)
    zsd_ref[...] = zz
    zdd_ref[...] = jnp.concatenate([zz[:, 8:], zz[:, 8:]], axis=1)


def _tc3(out2, b2, wsdT):
    return pl.pallas_call(
        _tc3_body,
        out_shape=[jax.ShapeDtypeStruct((N, 16), jnp.float32),
                   jax.ShapeDtypeStruct((N, 16), jnp.float32)],
        grid=(N // BN,),
        in_specs=[pl.BlockSpec((NC, BN, 48), lambda i: (0, i, 0)),
                  pl.BlockSpec((1, 32), lambda i: (0, 0)),
                  pl.BlockSpec((32, 16), lambda i: (0, 0))],
        out_specs=[pl.BlockSpec((BN, 16), lambda i: (i, 0)),
                   pl.BlockSpec((BN, 16), lambda i: (i, 0))],
    )(out2, b2, wsdT)


def _tc4a_body(ea_ref, zp_ref, waT_ref, bp1_ref, za_ref, ps_ref):
    za = (zp_ref[...]
          + jnp.dot(ea_ref[...], waT_ref[...],
                    preferred_element_type=jnp.float32)
          + bp1_ref[...])
    za_ref[...] = za
    ps_ref[0, 0, :] = jnp.sum(za, axis=0)
    ps_ref[0, 1, :] = jnp.sum(za * za, axis=0)


def _tc4a(ea, zp, waT, bp1_16):
    return pl.pallas_call(
        _tc4a_body,
        out_shape=[jax.ShapeDtypeStruct((E, 16), jnp.float32),
                   jax.ShapeDtypeStruct((NBE, 2, 16), jnp.float32)],
        grid=(NBE,),
        in_specs=[pl.BlockSpec((BE, 16), lambda i: (i, 0)),
                  pl.BlockSpec((BE, 16), lambda i: (i, 0)),
                  pl.BlockSpec((16, 16), lambda i: (0, 0)),
                  pl.BlockSpec((1, 16), lambda i: (0, 0))],
        out_specs=[pl.BlockSpec((BE, 16), lambda i: (i, 0)),
                   pl.BlockSpec((1, 2, 16), lambda i: (i, 0, 0))],
    )(ea, zp, waT, bp1_16)


def _tc4b_body(za_ref, ps_ref, g_ref, bt_ref, w2T_ref, bp2_ref, out_ref):
    tot = jnp.sum(ps_ref[...], axis=0)
    mean = tot[0] * (1.0 / E)
    var = tot[1] * (1.0 / E) - mean * mean
    zn = ((za_ref[...] - mean) / jnp.sqrt(var + 1e-5)
          * g_ref[...] + bt_ref[...])
    logits = (jnp.dot(zn, w2T_ref[...], preferred_element_type=jnp.float32)
              + bp2_ref[...])
    m = jnp.max(logits, axis=1, keepdims=True)
    s = logits - m
    out_ref[...] = s - jnp.log(jnp.sum(jnp.exp(s), axis=1, keepdims=True))


def _tc4b(za, ps, g16, bt16, w2T, bp2):
    return pl.pallas_call(
        _tc4b_body,
        out_shape=jax.ShapeDtypeStruct((E, 2), jnp.float32),
        grid=(NBE,),
        in_specs=[pl.BlockSpec((BE, 16), lambda i: (i, 0)),
                  pl.BlockSpec((NBE, 2, 16), lambda i: (0, 0, 0)),
                  pl.BlockSpec((1, 16), lambda i: (0, 0)),
                  pl.BlockSpec((1, 16), lambda i: (0, 0)),
                  pl.BlockSpec((16, 2), lambda i: (0, 0)),
                  pl.BlockSpec((1, 2), lambda i: (0, 0))],
        out_specs=pl.BlockSpec((BE, 2), lambda i: (i, 0)),
    )(za, ps, g16, bt16, w2T, bp2)


def _sc1_body(ht_h, elt_h, ert_h, src_h, dst_h, out_h,
              acc, sidx, didx, rows, elr, err, msg, exb):
    cid = lax.axis_index("c")
    sid = lax.axis_index("s")
    wid = cid * NS + sid
    zv = jnp.zeros((L,), jnp.float32)
    iota = lax.iota(jnp.int32, L)

    @plsc.parallel_loop(0, C, 1, unroll=4)
    def _zrow(r):
        for j in range(9):
            msg[r, pl.ds(j * L, L)] = zv
    for k in range(NPAD // NS // C):
        pltpu.sync_copy(msg, acc.at[pl.ds((sid * 8 + k) * C, C)])
    plsc.subcore_barrier()

    ebase = wid * EPW

    def chunk(c, _):
        base = ebase + c * C
        pltpu.sync_copy(src_h.at[pl.ds(base, C)], sidx)
        pltpu.sync_copy(dst_h.at[pl.ds(base, C)], didx)
        pltpu.sync_copy(ht_h.at[sidx], rows)
        pltpu.sync_copy(elt_h.at[sidx], elr)
        pltpu.sync_copy(ert_h.at[didx], err)

        def grp(g, _):
            rid = g * L + iota
            for h in range(8):
                hf = jnp.full((L,), h, jnp.int32)
                e = (plsc.load_gather(elr, [rid, hf])
                     + plsc.load_gather(err, [rid, hf]))
                e = jnp.maximum(e, 0.2 * e)
                exb[h, pl.ds(g * L, L)] = jnp.exp(e)
            return 0
        lax.fori_loop(0, C // L, grp, 0)

        @plsc.parallel_loop(0, C, 1, unroll=2)
        def _edge(e):
            ef = jnp.full((L,), e, jnp.int32)
            wv = plsc.load_gather(exb, [iota % 8, ef])
            msg[e, pl.ds(128, L)] = wv
            for j in range(8):
                w = plsc.load_gather(exb, [jnp.full((L,), j, jnp.int32), ef])
                msg[e, pl.ds(j * L, L)] = w * rows[e, pl.ds(j * L, L)]

        pltpu.sync_copy(msg, acc.at[didx], add=True)
        return 0
    lax.fori_loop(0, NCHUNKS, chunk, 0)

    plsc.subcore_barrier()
    rpw = NPAD // NS
    pltpu.sync_copy(acc.at[pl.ds(sid * rpw, rpw)],
                    out_h.at[cid, pl.ds(sid * rpw, rpw)])


_sc1 = pl.kernel(
    _sc1_body,
    out_type=jax.ShapeDtypeStruct((NC, NPAD, 144), jnp.float32),
    mesh=_MESH,
    compiler_params=_SC_PARAMS,
    scratch_types=[
        pltpu.VMEM_SHARED((NPAD, 144), jnp.float32),
        pltpu.VMEM((C,), jnp.int32),
        pltpu.VMEM((C,), jnp.int32),
        pltpu.VMEM((C, 128), jnp.float32),
        pltpu.VMEM((C, 16), jnp.float32),
        pltpu.VMEM((C, 16), jnp.float32),
        pltpu.VMEM((C, 144), jnp.float32),
        pltpu.VMEM((8, C), jnp.float32),
    ],
)


def _sc2_body(ht_h, elt_h, ert_h, src_h, dst_h, out_h,
              acc, sidx, didx, rows, elr, err, msg, exv):
    cid = lax.axis_index("c")
    sid = lax.axis_index("s")
    wid = cid * NS + sid
    zv = jnp.zeros((L,), jnp.float32)
    iota = lax.iota(jnp.int32, L)
    zf = jnp.zeros((L,), jnp.int32)

    @plsc.parallel_loop(0, C, 1, unroll=4)
    def _zrow(r):
        for j in range(3):
            msg[r, pl.ds(j * L, L)] = zv
    for k in range(NPAD // NS // C):
        pltpu.sync_copy(msg, acc.at[pl.ds((sid * 8 + k) * C, C)])
    plsc.subcore_barrier()

    ebase = wid * EPW

    def chunk(c, _):
        base = ebase + c * C
        pltpu.sync_copy(src_h.at[pl.ds(base, C)], sidx)
        pltpu.sync_copy(dst_h.at[pl.ds(base, C)], didx)
        pltpu.sync_copy(ht_h.at[sidx], rows)
        pltpu.sync_copy(elt_h.at[sidx], elr)
        pltpu.sync_copy(ert_h.at[didx], err)

        def grp(g, _):
            rid = g * L + iota
            e = (plsc.load_gather(elr, [rid, zf])
                 + plsc.load_gather(err, [rid, zf]))
            e = jnp.maximum(e, 0.2 * e)
            exv[pl.ds(g * L, L)] = jnp.exp(e)
            return 0
        lax.fori_loop(0, C // L, grp, 0)

        @plsc.parallel_loop(0, C, 1, unroll=4)
        def _edge(e):
            w = plsc.load_gather(exv, [jnp.full((L,), e, jnp.int32)])
            msg[e, pl.ds(0, L)] = w * rows[e, pl.ds(0, L)]
            msg[e, pl.ds(L, L)] = w * rows[e, pl.ds(L, L)]
            msg[e, pl.ds(2 * L, L)] = w

        pltpu.sync_copy(msg, acc.at[didx], add=True)
        return 0
    lax.fori_loop(0, NCHUNKS, chunk, 0)

    plsc.subcore_barrier()
    rpw = NPAD // NS
    pltpu.sync_copy(acc.at[pl.ds(sid * rpw, rpw)],
                    out_h.at[cid, pl.ds(sid * rpw, rpw)])


_sc2 = pl.kernel(
    _sc2_body,
    out_type=jax.ShapeDtypeStruct((NC, NPAD, 48), jnp.float32),
    mesh=_MESH,
    compiler_params=_SC_PARAMS,
    scratch_types=[
        pltpu.VMEM_SHARED((NPAD, 48), jnp.float32),
        pltpu.VMEM((C,), jnp.int32),
        pltpu.VMEM((C,), jnp.int32),
        pltpu.VMEM((C, 32), jnp.float32),
        pltpu.VMEM((C, 16), jnp.float32),
        pltpu.VMEM((C, 16), jnp.float32),
        pltpu.VMEM((C, 48), jnp.float32),
        pltpu.VMEM((C,), jnp.float32),
    ],
)


def _sc3_body(zsd_h, zdd_h, src_h, dst_h, out_h,
              sidx, didx, a, b, zout):
    cid = lax.axis_index("c")
    sid = lax.axis_index("s")
    wid = cid * NS + sid
    ebase = wid * EPW

    def chunk(c, _):
        base = ebase + c * C
        pltpu.sync_copy(src_h.at[pl.ds(base, C)], sidx)
        pltpu.sync_copy(dst_h.at[pl.ds(base, C)], didx)
        pltpu.sync_copy(zsd_h.at[sidx], a)
        pltpu.sync_copy(zdd_h.at[didx], b)

        @plsc.parallel_loop(0, C, 1, unroll=4)
        def _edge(e):
            zout[e, pl.ds(0, L)] = a[e, pl.ds(0, L)] + b[e, pl.ds(0, L)]

        pltpu.sync_copy(zout, out_h.at[pl.ds(base, C)])
        return 0
    lax.fori_loop(0, NCHUNKS, chunk, 0)


_sc3 = pl.kernel(
    _sc3_body,
    out_type=jax.ShapeDtypeStruct((E, 16), jnp.float32),
    mesh=_MESH,
    compiler_params=_SC_PARAMS,
    scratch_types=[
        pltpu.VMEM((C,), jnp.int32),
        pltpu.VMEM((C,), jnp.int32),
        pltpu.VMEM((C, 16), jnp.float32),
        pltpu.VMEM((C, 16), jnp.float32),
        pltpu.VMEM((C, 16), jnp.float32),
    ],
)


def kernel(n_feats, edge_index, edge_attr, W1, attn_l1, attn_r1, b1,
           W2, attn_l2, attn_r2, b2, Wp1, bp1, gamma, beta, Wp2, bp2):
    src = edge_index[0]
    dst = edge_index[1]
    f32 = jnp.float32

    alT = (jnp.eye(8, dtype=f32)[:, None, :]
           * attn_l1[:, :, None]).reshape(128, 8)
    arT = (jnp.eye(8, dtype=f32)[:, None, :]
           * attn_r1[:, :, None]).reshape(128, 8)
    ex8 = jnp.repeat(jnp.eye(8, dtype=f32), 16, axis=1)
    a2t = jnp.concatenate([attn_l2.reshape(32, 1),
                           attn_r2.reshape(32, 1)], axis=1)
    wa = Wp1[:, :16]
    wsdT = jnp.concatenate([Wp1[:, 16:48].T, Wp1[:, 48:80].T], axis=1)
    waT16 = jnp.concatenate([wa.T, jnp.zeros((16, 8), f32)], axis=1)
    bp1_16 = jnp.concatenate([bp1, jnp.zeros((8,), f32)]).reshape(1, 16)
    g16 = jnp.concatenate([gamma, jnp.ones((8,), f32)]).reshape(1, 16)
    bt16 = jnp.concatenate([beta, jnp.zeros((8,), f32)]).reshape(1, 16)
    w2T = jnp.concatenate([Wp2.T, jnp.zeros((8, 2), f32)], axis=0)
    bp2_r = bp2.reshape(1, 2)

    h, elt, ert = _tc1(n_feats, W1.T, alT, arT)
    out1 = _sc1(h, elt, ert, src, dst)
    ht2, elt2, ert2 = _tc2(out1, b1.reshape(1, 128), ex8, W2.T, a2t)

    out2 = _sc2(ht2, elt2, ert2, src, dst)
    zsd, zdd = _tc3(out2, b2.reshape(1, 32), wsdT)

    zp = _sc3(zsd, zdd, src, dst)
    za, ps = _tc4a(edge_attr, zp, waT16, bp1_16)
    return _tc4b(za, ps, g16, bt16, w2T, bp2_r)


# speedup vs baseline: 47.5467x; 1.8977x over previous
import jax
import jax.numpy as jnp
from jax import lax
from jax.experimental import pallas as pl
from jax.experimental.pallas import tpu as pltpu
from jax.experimental.pallas import tpu_sc as plsc

N = 10000
E = 320000
NC, NS, L = 2, 16, 16
NW = NC * NS
EPW = E // NW
C = 80
NCHUNKS = EPW // C
NPAD = 10240
BN = 2000
BE = 4000
NBE = E // BE

_SC_PARAMS = pltpu.CompilerParams(use_tc_tiling_on_sc=False,
                                  needs_layout_passes=False)
_MESH = plsc.VectorSubcoreMesh(core_axis_name="c", subcore_axis_name="s",
                               num_cores=NC, num_subcores=NS)


def _tc1_body(x_ref, w_ref, alT_ref, arT_ref, h_ref, elt_ref, ert_ref):
    h = jnp.dot(x_ref[...], w_ref[...], preferred_element_type=jnp.float32)
    h_ref[...] = h
    el = jnp.dot(h, alT_ref[...], preferred_element_type=jnp.float32)
    er = jnp.dot(h, arT_ref[...], preferred_element_type=jnp.float32)
    z = jnp.zeros_like(el)
    elt_ref[...] = jnp.concatenate([el, z], axis=1)
    ert_ref[...] = jnp.concatenate([er, z], axis=1)


def _tc1(x, w1t, alT, arT):
    return pl.pallas_call(
        _tc1_body,
        out_shape=[jax.ShapeDtypeStruct((N, 128), jnp.float32),
                   jax.ShapeDtypeStruct((N, 16), jnp.float32),
                   jax.ShapeDtypeStruct((N, 16), jnp.float32)],
        grid=(N // BN,),
        in_specs=[pl.BlockSpec((BN, 128), lambda i: (i, 0)),
                  pl.BlockSpec((128, 128), lambda i: (0, 0)),
                  pl.BlockSpec((128, 8), lambda i: (0, 0)),
                  pl.BlockSpec((128, 8), lambda i: (0, 0))],
        out_specs=[pl.BlockSpec((BN, 128), lambda i: (i, 0)),
                   pl.BlockSpec((BN, 16), lambda i: (i, 0)),
                   pl.BlockSpec((BN, 16), lambda i: (i, 0))],
    )(x, w1t, alT, arT)


def _tc2_body(o_ref, b1_ref, ex8_ref, w2t_ref, a2t_ref,
              ht2_ref, elt2_ref, ert2_ref):
    agg = o_ref[0, :, :128] + o_ref[1, :, :128]
    es = o_ref[0, :, 128:136] + o_ref[1, :, 128:136]
    es128 = jnp.dot(es, ex8_ref[...], preferred_element_type=jnp.float32)
    h1 = jnp.maximum(agg / (es128 + 1e-9) + b1_ref[...], 0.0)
    h2p = jnp.dot(h1, w2t_ref[...], preferred_element_type=jnp.float32)
    ea2 = jnp.dot(h2p, a2t_ref[...], preferred_element_type=jnp.float32)
    ht2_ref[...] = h2p
    z15 = jnp.zeros((h2p.shape[0], 15), jnp.float32)
    elt2_ref[...] = jnp.concatenate([ea2[:, 0:1], z15], axis=1)
    ert2_ref[...] = jnp.concatenate([ea2[:, 1:2], z15], axis=1)


def _tc2(out1, b1, ex8, w2t, a2t):
    return pl.pallas_call(
        _tc2_body,
        out_shape=[jax.ShapeDtypeStruct((N, 32), jnp.float32),
                   jax.ShapeDtypeStruct((N, 16), jnp.float32),
                   jax.ShapeDtypeStruct((N, 16), jnp.float32)],
        grid=(N // BN,),
        in_specs=[pl.BlockSpec((NC, BN, 144), lambda i: (0, i, 0)),
                  pl.BlockSpec((1, 128), lambda i: (0, 0)),
                  pl.BlockSpec((8, 128), lambda i: (0, 0)),
                  pl.BlockSpec((128, 32), lambda i: (0, 0)),
                  pl.BlockSpec((32, 2), lambda i: (0, 0))],
        out_specs=[pl.BlockSpec((BN, 32), lambda i: (i, 0)),
                   pl.BlockSpec((BN, 16), lambda i: (i, 0)),
                   pl.BlockSpec((BN, 16), lambda i: (i, 0))],
    )(out1, b1, ex8, w2t, a2t)


def _tc3_body(o_ref, b2_ref, wsdT_ref, zsd_ref, zdd_ref):
    agg = o_ref[0, :, :32] + o_ref[1, :, :32]
    es = o_ref[0, :, 32:33] + o_ref[1, :, 32:33]
    h2 = jnp.maximum(agg / (es + 1e-9) + b2_ref[...], 0.0)
    zz = jnp.dot(h2, wsdT_ref[...], preferred_element_type=jnp.float32)
    zsd_ref[...] = zz
    zdd_ref[...] = jnp.concatenate([zz[:, 8:], zz[:, 8:]], axis=1)


def _tc3(out2, b2, wsdT):
    return pl.pallas_call(
        _tc3_body,
        out_shape=[jax.ShapeDtypeStruct((N, 16), jnp.float32),
                   jax.ShapeDtypeStruct((N, 16), jnp.float32)],
        grid=(N // BN,),
        in_specs=[pl.BlockSpec((NC, BN, 48), lambda i: (0, i, 0)),
                  pl.BlockSpec((1, 32), lambda i: (0, 0)),
                  pl.BlockSpec((32, 16), lambda i: (0, 0))],
        out_specs=[pl.BlockSpec((BN, 16), lambda i: (i, 0)),
                   pl.BlockSpec((BN, 16), lambda i: (i, 0))],
    )(out2, b2, wsdT)


def _tc4a_body(ea_ref, zp_ref, waT_ref, bp1_ref, za_ref, ps_ref):
    za = (zp_ref[...]
          + jnp.dot(ea_ref[...], waT_ref[...],
                    preferred_element_type=jnp.float32)
          + bp1_ref[...])
    za_ref[...] = za
    ps_ref[0, 0, :] = jnp.sum(za, axis=0)
    ps_ref[0, 1, :] = jnp.sum(za * za, axis=0)


def _tc4a(ea, zp, waT, bp1_16):
    return pl.pallas_call(
        _tc4a_body,
        out_shape=[jax.ShapeDtypeStruct((E, 16), jnp.float32),
                   jax.ShapeDtypeStruct((NBE, 2, 16), jnp.float32)],
        grid=(NBE,),
        in_specs=[pl.BlockSpec((BE, 16), lambda i: (i, 0)),
                  pl.BlockSpec((BE, 16), lambda i: (i, 0)),
                  pl.BlockSpec((16, 16), lambda i: (0, 0)),
                  pl.BlockSpec((1, 16), lambda i: (0, 0))],
        out_specs=[pl.BlockSpec((BE, 16), lambda i: (i, 0)),
                   pl.BlockSpec((1, 2, 16), lambda i: (i, 0, 0))],
    )(ea, zp, waT, bp1_16)


def _tc4b_body(za_ref, ps_ref, g_ref, bt_ref, w2T_ref, bp2_ref, out_ref):
    tot = jnp.sum(ps_ref[...], axis=0)
    mean = tot[0] * (1.0 / E)
    var = tot[1] * (1.0 / E) - mean * mean
    zn = ((za_ref[...] - mean) / jnp.sqrt(var + 1e-5)
          * g_ref[...] + bt_ref[...])
    logits = (jnp.dot(zn, w2T_ref[...], preferred_element_type=jnp.float32)
              + bp2_ref[...])
    m = jnp.max(logits, axis=1, keepdims=True)
    s = logits - m
    out_ref[...] = s - jnp.log(jnp.sum(jnp.exp(s), axis=1, keepdims=True))


def _tc4b(za, ps, g16, bt16, w2T, bp2):
    return pl.pallas_call(
        _tc4b_body,
        out_shape=jax.ShapeDtypeStruct((E, 2), jnp.float32),
        grid=(NBE,),
        in_specs=[pl.BlockSpec((BE, 16), lambda i: (i, 0)),
                  pl.BlockSpec((NBE, 2, 16), lambda i: (0, 0, 0)),
                  pl.BlockSpec((1, 16), lambda i: (0, 0)),
                  pl.BlockSpec((1, 16), lambda i: (0, 0)),
                  pl.BlockSpec((16, 2), lambda i: (0, 0)),
                  pl.BlockSpec((1, 2), lambda i: (0, 0))],
        out_specs=pl.BlockSpec((BE, 2), lambda i: (i, 0)),
    )(za, ps, g16, bt16, w2T, bp2)


C1 = 40
NCH1 = EPW // C1
SEG = 50
NSEG = NCH1 // SEG


def _sc1_body(ht_h, elt_h, ert_h, src2_h, dst2_h, out_h,
              acc, sslab, dslab, rows0, rows1, elr0, elr1,
              err0, err1, msg0, msg1, exb, sg0, sg1, ss0, ss1):
    cid = lax.axis_index("c")
    sid = lax.axis_index("s")
    wid = cid * NS + sid
    zv = jnp.zeros((L,), jnp.float32)
    iota = lax.iota(jnp.int32, L)
    rows = (rows0, rows1)
    elr = (elr0, elr1)
    err = (err0, err1)
    msg = (msg0, msg1)
    sg = (sg0, sg1)
    ss = (ss0, ss1)

    def issue_gathers(c, b):
        pltpu.async_copy(ht_h.at[sslab.at[c]], rows[b], sg[b])
        pltpu.async_copy(elt_h.at[sslab.at[c]], elr[b].at[pl.ds(0, C1)], sg[b])
        pltpu.async_copy(ert_h.at[dslab.at[c]], err[b].at[pl.ds(0, C1)], sg[b])

    def wait_gathers(b):
        pltpu.make_async_copy(ht_h.at[sslab.at[0]], rows[b], sg[b]).wait()
        pltpu.make_async_copy(elt_h.at[sslab.at[0]],
                              elr[b].at[pl.ds(0, C1)], sg[b]).wait()
        pltpu.make_async_copy(ert_h.at[dslab.at[0]],
                              err[b].at[pl.ds(0, C1)], sg[b]).wait()

    def drain_scatter(b):
        pltpu.make_async_copy(msg[b], acc.at[dslab.at[0]], ss[b]).wait()

    def compute(c, b):
        def grp(g, _):
            rid = g * L + iota
            for h in range(8):
                hf = jnp.full((L,), h, jnp.int32)
                e = (plsc.load_gather(elr[b], [rid, hf])
                     + plsc.load_gather(err[b], [rid, hf]))
                e = jnp.maximum(e, 0.2 * e)
                exb[h, pl.ds(g * L, L)] = jnp.exp(e)
            return 0
        lax.fori_loop(0, 3, grp, 0)

        @pl.when(c >= 2)
        def _():
            drain_scatter(b)

        @plsc.parallel_loop(0, C1, 1, unroll=2)
        def _edge(e):
            ef = jnp.full((L,), e, jnp.int32)
            wv = plsc.load_gather(exb, [iota % 8, ef])
            msg[b][e, pl.ds(128, L)] = wv
            for j in range(8):
                w = plsc.load_gather(exb, [jnp.full((L,), j, jnp.int32), ef])
                msg[b][e, pl.ds(j * L, L)] = w * rows[b][e, pl.ds(j * L, L)]

        pltpu.async_copy(msg[b], acc.at[dslab.at[c]], ss[b], add=True)

    @plsc.parallel_loop(0, C1, 1, unroll=4)
    def _zrow(r):
        for j in range(9):
            msg0[r, pl.ds(j * L, L)] = zv
    for k in range(NPAD // NS // C1):
        pltpu.sync_copy(msg0, acc.at[pl.ds((sid * 16 + k) * C1, C1)])
    plsc.subcore_barrier()

    def segment(s, _):
        @pl.when(s > 0)
        def _():
            drain_scatter(0)
            drain_scatter(1)
        cbase = wid * NCH1 + s * SEG
        pltpu.sync_copy(src2_h.at[pl.ds(cbase, SEG)], sslab)
        pltpu.sync_copy(dst2_h.at[pl.ds(cbase, SEG)], dslab)
        issue_gathers(0, 0)
        issue_gathers(1, 1)

        def pair(it, _):
            for b in range(2):
                c = 2 * it + b
                wait_gathers(b)
                compute(c, b)

                @pl.when(c < SEG - 2)
                def _():
                    issue_gathers(c + 2, b)
            return 0
        lax.fori_loop(0, SEG // 2, pair, 0)
        return 0
    lax.fori_loop(0, NSEG, segment, 0)

    drain_scatter(0)
    drain_scatter(1)
    plsc.subcore_barrier()
    rpw = NPAD // NS
    pltpu.sync_copy(acc.at[pl.ds(sid * rpw, rpw)],
                    out_h.at[cid, pl.ds(sid * rpw, rpw)])


_sc1 = pl.kernel(
    _sc1_body,
    out_type=jax.ShapeDtypeStruct((NC, NPAD, 144), jnp.float32),
    mesh=_MESH,
    compiler_params=_SC_PARAMS,
    scratch_types=[
        pltpu.VMEM_SHARED((NPAD, 144), jnp.float32),
        pltpu.VMEM((SEG, C1), jnp.int32),
        pltpu.VMEM((SEG, C1), jnp.int32),
        pltpu.VMEM((C1, 128), jnp.float32),
        pltpu.VMEM((C1, 128), jnp.float32),
        pltpu.VMEM((48, 16), jnp.float32),
        pltpu.VMEM((48, 16), jnp.float32),
        pltpu.VMEM((48, 16), jnp.float32),
        pltpu.VMEM((48, 16), jnp.float32),
        pltpu.VMEM((C1, 144), jnp.float32),
        pltpu.VMEM((C1, 144), jnp.float32),
        pltpu.VMEM((8, 48), jnp.float32),
        pltpu.SemaphoreType.DMA,
        pltpu.SemaphoreType.DMA,
        pltpu.SemaphoreType.DMA,
        pltpu.SemaphoreType.DMA,
    ],
)


def _sc2_body(ht_h, elt_h, ert_h, src2_h, dst2_h, out_h,
              acc, sidx_all, didx_all, rows0, rows1, elr0, elr1,
              err0, err1, msg0, msg1, exv, sg0, sg1, ss0, ss1):
    cid = lax.axis_index("c")
    sid = lax.axis_index("s")
    wid = cid * NS + sid
    zv = jnp.zeros((L,), jnp.float32)
    iota = lax.iota(jnp.int32, L)
    zf = jnp.zeros((L,), jnp.int32)
    rows = (rows0, rows1)
    elr = (elr0, elr1)
    err = (err0, err1)
    msg = (msg0, msg1)
    sg = (sg0, sg1)
    ss = (ss0, ss1)

    cbase = wid * NCHUNKS
    pltpu.sync_copy(src2_h.at[pl.ds(cbase, NCHUNKS)], sidx_all)
    pltpu.sync_copy(dst2_h.at[pl.ds(cbase, NCHUNKS)], didx_all)

    def issue_gathers(c, b):
        pltpu.async_copy(ht_h.at[sidx_all.at[c]], rows[b], sg[b])
        pltpu.async_copy(elt_h.at[sidx_all.at[c]], elr[b], sg[b])
        pltpu.async_copy(ert_h.at[didx_all.at[c]], err[b], sg[b])

    def wait_gathers(b):
        pltpu.make_async_copy(ht_h.at[sidx_all.at[0]], rows[b], sg[b]).wait()
        pltpu.make_async_copy(elt_h.at[sidx_all.at[0]], elr[b], sg[b]).wait()
        pltpu.make_async_copy(ert_h.at[didx_all.at[0]], err[b], sg[b]).wait()

    def compute(c, b):
        def grp(g, _):
            rid = g * L + iota
            e = (plsc.load_gather(elr[b], [rid, zf])
                 + plsc.load_gather(err[b], [rid, zf]))
            e = jnp.maximum(e, 0.2 * e)
            exv[pl.ds(g * L, L)] = jnp.exp(e)
            return 0
        lax.fori_loop(0, C // L, grp, 0)

        @pl.when(c >= 2)
        def _():
            pltpu.make_async_copy(msg[b], acc.at[didx_all.at[c]],
                                  ss[b]).wait()

        @plsc.parallel_loop(0, C, 1, unroll=4)
        def _edge(e):
            w = plsc.load_gather(exv, [jnp.full((L,), e, jnp.int32)])
            msg[b][e, pl.ds(0, L)] = w * rows[b][e, pl.ds(0, L)]
            msg[b][e, pl.ds(L, L)] = w * rows[b][e, pl.ds(L, L)]
            msg[b][e, pl.ds(2 * L, L)] = w

        pltpu.async_copy(msg[b], acc.at[didx_all.at[c]], ss[b], add=True)

    @plsc.parallel_loop(0, C, 1, unroll=4)
    def _zrow(r):
        for j in range(3):
            msg0[r, pl.ds(j * L, L)] = zv
    for k in range(NPAD // NS // C):
        pltpu.sync_copy(msg0, acc.at[pl.ds((sid * 8 + k) * C, C)])
    plsc.subcore_barrier()

    issue_gathers(0, 0)
    issue_gathers(1, 1)

    def pair(it, _):
        for b in range(2):
            c = 2 * it + b
            wait_gathers(b)
            compute(c, b)

            @pl.when(c < NCHUNKS - 2)
            def _():
                issue_gathers(c + 2, b)
        return 0
    lax.fori_loop(0, (NCHUNKS - 1) // 2, pair, 0)

    wait_gathers(0)
    compute(NCHUNKS - 1, 0)
    pltpu.make_async_copy(msg1, acc.at[didx_all.at[0]], ss1).wait()
    pltpu.make_async_copy(msg0, acc.at[didx_all.at[0]], ss0).wait()

    plsc.subcore_barrier()
    rpw = NPAD // NS
    pltpu.sync_copy(acc.at[pl.ds(sid * rpw, rpw)],
                    out_h.at[cid, pl.ds(sid * rpw, rpw)])


_sc2 = pl.kernel(
    _sc2_body,
    out_type=jax.ShapeDtypeStruct((NC, NPAD, 48), jnp.float32),
    mesh=_MESH,
    compiler_params=_SC_PARAMS,
    scratch_types=[
        pltpu.VMEM_SHARED((NPAD, 48), jnp.float32),
        pltpu.VMEM((NCHUNKS, C), jnp.int32),
        pltpu.VMEM((NCHUNKS, C), jnp.int32),
        pltpu.VMEM((C, 32), jnp.float32),
        pltpu.VMEM((C, 32), jnp.float32),
        pltpu.VMEM((C, 16), jnp.float32),
        pltpu.VMEM((C, 16), jnp.float32),
        pltpu.VMEM((C, 16), jnp.float32),
        pltpu.VMEM((C, 16), jnp.float32),
        pltpu.VMEM((C, 48), jnp.float32),
        pltpu.VMEM((C, 48), jnp.float32),
        pltpu.VMEM((C,), jnp.float32),
        pltpu.SemaphoreType.DMA,
        pltpu.SemaphoreType.DMA,
        pltpu.SemaphoreType.DMA,
        pltpu.SemaphoreType.DMA,
    ],
)


def _sc3_body(zsd_h, zdd_h, src2_h, dst2_h, out_h,
              sidx_all, didx_all, ga0, ga1, gb0, gb1, zout0, zout1,
              sg0, sg1, sw0, sw1):
    cid = lax.axis_index("c")
    sid = lax.axis_index("s")
    wid = cid * NS + sid
    ga = (ga0, ga1)
    gb = (gb0, gb1)
    zout = (zout0, zout1)
    sg = (sg0, sg1)
    sw = (sw0, sw1)
    ebase = wid * EPW

    cbase = wid * NCHUNKS
    pltpu.sync_copy(src2_h.at[pl.ds(cbase, NCHUNKS)], sidx_all)
    pltpu.sync_copy(dst2_h.at[pl.ds(cbase, NCHUNKS)], didx_all)

    def issue_gathers(c, b):
        pltpu.async_copy(zsd_h.at[sidx_all.at[c]], ga[b], sg[b])
        pltpu.async_copy(zdd_h.at[didx_all.at[c]], gb[b], sg[b])

    def wait_gathers(b):
        pltpu.make_async_copy(zsd_h.at[sidx_all.at[0]], ga[b], sg[b]).wait()
        pltpu.make_async_copy(zdd_h.at[didx_all.at[0]], gb[b], sg[b]).wait()

    def compute(c, b):
        base = ebase + c * C

        @pl.when(c >= 2)
        def _():
            pltpu.make_async_copy(zout[b], out_h.at[pl.ds(base, C)],
                                  sw[b]).wait()

        @plsc.parallel_loop(0, C, 1, unroll=4)
        def _edge(e):
            zout[b][e, pl.ds(0, L)] = (ga[b][e, pl.ds(0, L)]
                                       + gb[b][e, pl.ds(0, L)])

        pltpu.async_copy(zout[b], out_h.at[pl.ds(base, C)], sw[b])

    issue_gathers(0, 0)
    issue_gathers(1, 1)

    def pair(it, _):
        for b in range(2):
            c = 2 * it + b
            wait_gathers(b)
            compute(c, b)

            @pl.when(c < NCHUNKS - 2)
            def _():
                issue_gathers(c + 2, b)
        return 0
    lax.fori_loop(0, (NCHUNKS - 1) // 2, pair, 0)

    wait_gathers(0)
    compute(NCHUNKS - 1, 0)
    pltpu.make_async_copy(zout1, out_h.at[pl.ds(ebase, C)], sw1).wait()
    pltpu.make_async_copy(zout0, out_h.at[pl.ds(ebase, C)], sw0).wait()


_sc3 = pl.kernel(
    _sc3_body,
    out_type=jax.ShapeDtypeStruct((E, 16), jnp.float32),
    mesh=_MESH,
    compiler_params=_SC_PARAMS,
    scratch_types=[
        pltpu.VMEM((NCHUNKS, C), jnp.int32),
        pltpu.VMEM((NCHUNKS, C), jnp.int32),
        pltpu.VMEM((C, 16), jnp.float32),
        pltpu.VMEM((C, 16), jnp.float32),
        pltpu.VMEM((C, 16), jnp.float32),
        pltpu.VMEM((C, 16), jnp.float32),
        pltpu.VMEM((C, 16), jnp.float32),
        pltpu.VMEM((C, 16), jnp.float32),
        pltpu.SemaphoreType.DMA,
        pltpu.SemaphoreType.DMA,
        pltpu.SemaphoreType.DMA,
        pltpu.SemaphoreType.DMA,
    ],
)


def kernel(n_feats, edge_index, edge_attr, W1, attn_l1, attn_r1, b1,
           W2, attn_l2, attn_r2, b2, Wp1, bp1, gamma, beta, Wp2, bp2):
    src = edge_index[0].reshape(E // C, C)
    dst = edge_index[1].reshape(E // C, C)
    src1 = edge_index[0].reshape(E // C1, C1)
    dst1 = edge_index[1].reshape(E // C1, C1)
    f32 = jnp.float32

    alT = (jnp.eye(8, dtype=f32)[:, None, :]
           * attn_l1[:, :, None]).reshape(128, 8)
    arT = (jnp.eye(8, dtype=f32)[:, None, :]
           * attn_r1[:, :, None]).reshape(128, 8)
    ex8 = jnp.repeat(jnp.eye(8, dtype=f32), 16, axis=1)
    a2t = jnp.concatenate([attn_l2.reshape(32, 1),
                           attn_r2.reshape(32, 1)], axis=1)
    wa = Wp1[:, :16]
    wsdT = jnp.concatenate([Wp1[:, 16:48].T, Wp1[:, 48:80].T], axis=1)
    waT16 = jnp.concatenate([wa.T, jnp.zeros((16, 8), f32)], axis=1)
    bp1_16 = jnp.concatenate([bp1, jnp.zeros((8,), f32)]).reshape(1, 16)
    g16 = jnp.concatenate([gamma, jnp.ones((8,), f32)]).reshape(1, 16)
    bt16 = jnp.concatenate([beta, jnp.zeros((8,), f32)]).reshape(1, 16)
    w2T = jnp.concatenate([Wp2.T, jnp.zeros((8, 2), f32)], axis=0)
    bp2_r = bp2.reshape(1, 2)

    h, elt, ert = _tc1(n_feats, W1.T, alT, arT)
    out1 = _sc1(h, elt, ert, src1, dst1)
    ht2, elt2, ert2 = _tc2(out1, b1.reshape(1, 128), ex8, W2.T, a2t)

    out2 = _sc2(ht2, elt2, ert2, src, dst)
    zsd, zdd = _tc3(out2, b2.reshape(1, 32), wsdT)

    zp = _sc3(zsd, zdd, src, dst)
    za, ps = _tc4a(edge_attr, zp, waT16, bp1_16)
    return _tc4b(za, ps, g16, bt16, w2T, bp2_r)


# speedup vs baseline: 49.5268x; 1.0416x over previous
import jax
import jax.numpy as jnp
from jax import lax
from jax.experimental import pallas as pl
from jax.experimental.pallas import tpu as pltpu
from jax.experimental.pallas import tpu_sc as plsc

N = 10000
E = 320000
NC, NS, L = 2, 16, 16
NW = NC * NS
EPW = E // NW
C = 80
NCHUNKS = EPW // C
NPAD = 10240
BN = 2000
BE = 4000
NBE = E // BE

_SC_PARAMS = pltpu.CompilerParams(use_tc_tiling_on_sc=False,
                                  needs_layout_passes=False)
_MESH = plsc.VectorSubcoreMesh(core_axis_name="c", subcore_axis_name="s",
                               num_cores=NC, num_subcores=NS)


def _tc1_body(x_ref, w_ref, alT_ref, arT_ref, h_ref, elt_ref, ert_ref):
    h = jnp.dot(x_ref[...], w_ref[...], preferred_element_type=jnp.float32)
    h_ref[...] = h
    el = jnp.dot(h, alT_ref[...], preferred_element_type=jnp.float32)
    er = jnp.dot(h, arT_ref[...], preferred_element_type=jnp.float32)
    z = jnp.zeros_like(el)
    elt_ref[...] = jnp.concatenate([el, z], axis=1)
    ert_ref[...] = jnp.concatenate([er, z], axis=1)


def _tc1(x, w1t, alT, arT):
    return pl.pallas_call(
        _tc1_body,
        out_shape=[jax.ShapeDtypeStruct((N, 128), jnp.float32),
                   jax.ShapeDtypeStruct((N, 16), jnp.float32),
                   jax.ShapeDtypeStruct((N, 16), jnp.float32)],
        grid=(N // BN,),
        in_specs=[pl.BlockSpec((BN, 128), lambda i: (i, 0)),
                  pl.BlockSpec((128, 128), lambda i: (0, 0)),
                  pl.BlockSpec((128, 8), lambda i: (0, 0)),
                  pl.BlockSpec((128, 8), lambda i: (0, 0))],
        out_specs=[pl.BlockSpec((BN, 128), lambda i: (i, 0)),
                   pl.BlockSpec((BN, 16), lambda i: (i, 0)),
                   pl.BlockSpec((BN, 16), lambda i: (i, 0))],
    )(x, w1t, alT, arT)


def _tc2_body(o_ref, b1_ref, ex8_ref, w2t_ref, a2t_ref,
              ht2_ref, elt2_ref, ert2_ref):
    agg = o_ref[0, :, :128] + o_ref[1, :, :128]
    es = o_ref[0, :, 128:136] + o_ref[1, :, 128:136]
    es128 = jnp.dot(es, ex8_ref[...], preferred_element_type=jnp.float32)
    h1 = jnp.maximum(agg / (es128 + 1e-9) + b1_ref[...], 0.0)
    h2p = jnp.dot(h1, w2t_ref[...], preferred_element_type=jnp.float32)
    ea2 = jnp.dot(h2p, a2t_ref[...], preferred_element_type=jnp.float32)
    ht2_ref[...] = h2p
    z15 = jnp.zeros((h2p.shape[0], 15), jnp.float32)
    elt2_ref[...] = jnp.concatenate([ea2[:, 0:1], z15], axis=1)
    ert2_ref[...] = jnp.concatenate([ea2[:, 1:2], z15], axis=1)


def _tc2(out1, b1, ex8, w2t, a2t):
    return pl.pallas_call(
        _tc2_body,
        out_shape=[jax.ShapeDtypeStruct((N, 32), jnp.float32),
                   jax.ShapeDtypeStruct((N, 16), jnp.float32),
                   jax.ShapeDtypeStruct((N, 16), jnp.float32)],
        grid=(N // BN,),
        in_specs=[pl.BlockSpec((NC, BN, 144), lambda i: (0, i, 0)),
                  pl.BlockSpec((1, 128), lambda i: (0, 0)),
                  pl.BlockSpec((8, 128), lambda i: (0, 0)),
                  pl.BlockSpec((128, 32), lambda i: (0, 0)),
                  pl.BlockSpec((32, 2), lambda i: (0, 0))],
        out_specs=[pl.BlockSpec((BN, 32), lambda i: (i, 0)),
                   pl.BlockSpec((BN, 16), lambda i: (i, 0)),
                   pl.BlockSpec((BN, 16), lambda i: (i, 0))],
    )(out1, b1, ex8, w2t, a2t)


def _tc3_body(o_ref, b2_ref, wsdT_ref, zsd_ref, zdd_ref):
    agg = o_ref[0, :, :32] + o_ref[1, :, :32]
    es = o_ref[0, :, 32:33] + o_ref[1, :, 32:33]
    h2 = jnp.maximum(agg / (es + 1e-9) + b2_ref[...], 0.0)
    zz = jnp.dot(h2, wsdT_ref[...], preferred_element_type=jnp.float32)
    zsd_ref[...] = zz
    zdd_ref[...] = jnp.concatenate([zz[:, 8:], zz[:, 8:]], axis=1)


def _tc3(out2, b2, wsdT):
    return pl.pallas_call(
        _tc3_body,
        out_shape=[jax.ShapeDtypeStruct((N, 16), jnp.float32),
                   jax.ShapeDtypeStruct((N, 16), jnp.float32)],
        grid=(N // BN,),
        in_specs=[pl.BlockSpec((NC, BN, 48), lambda i: (0, i, 0)),
                  pl.BlockSpec((1, 32), lambda i: (0, 0)),
                  pl.BlockSpec((32, 16), lambda i: (0, 0))],
        out_specs=[pl.BlockSpec((BN, 16), lambda i: (i, 0)),
                   pl.BlockSpec((BN, 16), lambda i: (i, 0))],
    )(out2, b2, wsdT)


def _tc4a_body(ea_ref, zp_ref, waT_ref, bp1_ref, za_ref, ps_ref):
    za = (zp_ref[...]
          + jnp.dot(ea_ref[...], waT_ref[...],
                    preferred_element_type=jnp.float32)
          + bp1_ref[...])
    za_ref[...] = za
    ps_ref[0, 0, :] = jnp.sum(za, axis=0)
    ps_ref[0, 1, :] = jnp.sum(za * za, axis=0)


def _tc4a(ea, zp, waT, bp1_16):
    return pl.pallas_call(
        _tc4a_body,
        out_shape=[jax.ShapeDtypeStruct((E, 16), jnp.float32),
                   jax.ShapeDtypeStruct((NBE, 2, 16), jnp.float32)],
        grid=(NBE,),
        in_specs=[pl.BlockSpec((BE, 16), lambda i: (i, 0)),
                  pl.BlockSpec((BE, 16), lambda i: (i, 0)),
                  pl.BlockSpec((16, 16), lambda i: (0, 0)),
                  pl.BlockSpec((1, 16), lambda i: (0, 0))],
        out_specs=[pl.BlockSpec((BE, 16), lambda i: (i, 0)),
                   pl.BlockSpec((1, 2, 16), lambda i: (i, 0, 0))],
    )(ea, zp, waT, bp1_16)


def _tc4b_body(za_ref, ps_ref, g_ref, bt_ref, w2T_ref, bp2_ref, out_ref):
    tot = jnp.sum(ps_ref[...], axis=0)
    mean = tot[0] * (1.0 / E)
    var = tot[1] * (1.0 / E) - mean * mean
    zn = ((za_ref[...] - mean) / jnp.sqrt(var + 1e-5)
          * g_ref[...] + bt_ref[...])
    logits = (jnp.dot(zn, w2T_ref[...], preferred_element_type=jnp.float32)
              + bp2_ref[...])
    m = jnp.max(logits, axis=1, keepdims=True)
    s = logits - m
    out_ref[...] = s - jnp.log(jnp.sum(jnp.exp(s), axis=1, keepdims=True))


def _tc4b(za, ps, g16, bt16, w2T, bp2):
    return pl.pallas_call(
        _tc4b_body,
        out_shape=jax.ShapeDtypeStruct((E, 2), jnp.float32),
        grid=(NBE,),
        in_specs=[pl.BlockSpec((BE, 16), lambda i: (i, 0)),
                  pl.BlockSpec((NBE, 2, 16), lambda i: (0, 0, 0)),
                  pl.BlockSpec((1, 16), lambda i: (0, 0)),
                  pl.BlockSpec((1, 16), lambda i: (0, 0)),
                  pl.BlockSpec((16, 2), lambda i: (0, 0)),
                  pl.BlockSpec((1, 2), lambda i: (0, 0))],
        out_specs=pl.BlockSpec((BE, 2), lambda i: (i, 0)),
    )(za, ps, g16, bt16, w2T, bp2)


C1 = 40
NCH1 = EPW // C1
SEG = 50
NSEG = NCH1 // SEG


def _sc1_body(ht_h, elt_h, ert_h, src2_h, dst2_h, out_h,
              acc, sslab, dslab, rows0, rows1, elr0, elr1,
              err0, err1, msg0, msg1, exb, sg0, sg1, ss0, ss1):
    cid = lax.axis_index("c")
    sid = lax.axis_index("s")
    wid = cid * NS + sid
    zv = jnp.zeros((L,), jnp.float32)
    iota = lax.iota(jnp.int32, L)
    rows = (rows0, rows1)
    elr = (elr0, elr1)
    err = (err0, err1)
    msg = (msg0, msg1)
    sg = (sg0, sg1)
    ss = (ss0, ss1)

    def issue_gathers(c, b):
        pltpu.async_copy(ht_h.at[sslab.at[c]], rows[b], sg[b])
        pltpu.async_copy(elt_h.at[sslab.at[c]], elr[b].at[pl.ds(0, C1)], sg[b])
        pltpu.async_copy(ert_h.at[dslab.at[c]], err[b].at[pl.ds(0, C1)], sg[b])

    def wait_gathers(b):
        pltpu.make_async_copy(ht_h.at[sslab.at[0]], rows[b], sg[b]).wait()
        pltpu.make_async_copy(elt_h.at[sslab.at[0]],
                              elr[b].at[pl.ds(0, C1)], sg[b]).wait()
        pltpu.make_async_copy(ert_h.at[dslab.at[0]],
                              err[b].at[pl.ds(0, C1)], sg[b]).wait()

    def drain_scatter(b):
        pltpu.make_async_copy(msg[b], acc.at[dslab.at[0]], ss[b]).wait()

    def compute(c, b):
        def grp(g, _):
            rid = g * L + iota
            for h in range(8):
                hf = jnp.full((L,), h, jnp.int32)
                e = (plsc.load_gather(elr[b], [rid, hf])
                     + plsc.load_gather(err[b], [rid, hf]))
                e = jnp.maximum(e, 0.2 * e)
                exb[h, pl.ds(g * L, L)] = jnp.exp(e)
            return 0
        lax.fori_loop(0, 3, grp, 0)

        @pl.when(c >= 2)
        def _():
            drain_scatter(b)

        @plsc.parallel_loop(0, C1, 1, unroll=2)
        def _edge(e):
            ef = jnp.full((L,), e, jnp.int32)
            wv = plsc.load_gather(exb, [iota % 8, ef])
            msg[b][e, pl.ds(128, L)] = wv
            for j in range(8):
                w = wv[jnp.full((L,), j, jnp.int32)]
                msg[b][e, pl.ds(j * L, L)] = w * rows[b][e, pl.ds(j * L, L)]

        pltpu.async_copy(msg[b], acc.at[dslab.at[c]], ss[b], add=True)

    @plsc.parallel_loop(0, C1, 1, unroll=4)
    def _zrow(r):
        for j in range(9):
            msg0[r, pl.ds(j * L, L)] = zv
    for k in range(NPAD // NS // C1):
        pltpu.sync_copy(msg0, acc.at[pl.ds((sid * 16 + k) * C1, C1)])
    plsc.subcore_barrier()

    def segment(s, _):
        @pl.when(s > 0)
        def _():
            drain_scatter(0)
            drain_scatter(1)
        cbase = wid * NCH1 + s * SEG
        pltpu.sync_copy(src2_h.at[pl.ds(cbase, SEG)], sslab)
        pltpu.sync_copy(dst2_h.at[pl.ds(cbase, SEG)], dslab)
        issue_gathers(0, 0)
        issue_gathers(1, 1)

        def pair(it, _):
            for b in range(2):
                c = 2 * it + b
                wait_gathers(b)
                compute(c, b)

                @pl.when(c < SEG - 2)
                def _():
                    issue_gathers(c + 2, b)
            return 0
        lax.fori_loop(0, SEG // 2, pair, 0)
        return 0
    lax.fori_loop(0, NSEG, segment, 0)

    drain_scatter(0)
    drain_scatter(1)
    plsc.subcore_barrier()
    rpw = NPAD // NS
    pltpu.sync_copy(acc.at[pl.ds(sid * rpw, rpw)],
                    out_h.at[cid, pl.ds(sid * rpw, rpw)])


_sc1 = pl.kernel(
    _sc1_body,
    out_type=jax.ShapeDtypeStruct((NC, NPAD, 144), jnp.float32),
    mesh=_MESH,
    compiler_params=_SC_PARAMS,
    scratch_types=[
        pltpu.VMEM_SHARED((NPAD, 144), jnp.float32),
        pltpu.VMEM((SEG, C1), jnp.int32),
        pltpu.VMEM((SEG, C1), jnp.int32),
        pltpu.VMEM((C1, 128), jnp.float32),
        pltpu.VMEM((C1, 128), jnp.float32),
        pltpu.VMEM((48, 16), jnp.float32),
        pltpu.VMEM((48, 16), jnp.float32),
        pltpu.VMEM((48, 16), jnp.float32),
        pltpu.VMEM((48, 16), jnp.float32),
        pltpu.VMEM((C1, 144), jnp.float32),
        pltpu.VMEM((C1, 144), jnp.float32),
        pltpu.VMEM((8, 48), jnp.float32),
        pltpu.SemaphoreType.DMA,
        pltpu.SemaphoreType.DMA,
        pltpu.SemaphoreType.DMA,
        pltpu.SemaphoreType.DMA,
    ],
)


def _sc2_body(ht_h, elt_h, ert_h, src2_h, dst2_h, out_h,
              acc, sidx_all, didx_all, rows0, rows1, elr0, elr1,
              err0, err1, msg0, msg1, exv, sg0, sg1, ss0, ss1):
    cid = lax.axis_index("c")
    sid = lax.axis_index("s")
    wid = cid * NS + sid
    zv = jnp.zeros((L,), jnp.float32)
    iota = lax.iota(jnp.int32, L)
    zf = jnp.zeros((L,), jnp.int32)
    rows = (rows0, rows1)
    elr = (elr0, elr1)
    err = (err0, err1)
    msg = (msg0, msg1)
    sg = (sg0, sg1)
    ss = (ss0, ss1)

    cbase = wid * NCHUNKS
    pltpu.sync_copy(src2_h.at[pl.ds(cbase, NCHUNKS)], sidx_all)
    pltpu.sync_copy(dst2_h.at[pl.ds(cbase, NCHUNKS)], didx_all)

    def issue_gathers(c, b):
        pltpu.async_copy(ht_h.at[sidx_all.at[c]], rows[b], sg[b])
        pltpu.async_copy(elt_h.at[sidx_all.at[c]], elr[b], sg[b])
        pltpu.async_copy(ert_h.at[didx_all.at[c]], err[b], sg[b])

    def wait_gathers(b):
        pltpu.make_async_copy(ht_h.at[sidx_all.at[0]], rows[b], sg[b]).wait()
        pltpu.make_async_copy(elt_h.at[sidx_all.at[0]], elr[b], sg[b]).wait()
        pltpu.make_async_copy(ert_h.at[didx_all.at[0]], err[b], sg[b]).wait()

    def compute(c, b):
        def grp(g, _):
            rid = g * L + iota
            e = (plsc.load_gather(elr[b], [rid, zf])
                 + plsc.load_gather(err[b], [rid, zf]))
            e = jnp.maximum(e, 0.2 * e)
            exv[pl.ds(g * L, L)] = jnp.exp(e)
            return 0
        lax.fori_loop(0, C // L, grp, 0)

        @pl.when(c >= 2)
        def _():
            pltpu.make_async_copy(msg[b], acc.at[didx_all.at[c]],
                                  ss[b]).wait()

        @plsc.parallel_loop(0, C, 1, unroll=4)
        def _edge(e):
            w = plsc.load_gather(exv, [jnp.full((L,), e, jnp.int32)])
            msg[b][e, pl.ds(0, L)] = w * rows[b][e, pl.ds(0, L)]
            msg[b][e, pl.ds(L, L)] = w * rows[b][e, pl.ds(L, L)]
            msg[b][e, pl.ds(2 * L, L)] = w

        pltpu.async_copy(msg[b], acc.at[didx_all.at[c]], ss[b], add=True)

    @plsc.parallel_loop(0, C, 1, unroll=4)
    def _zrow(r):
        for j in range(3):
            msg0[r, pl.ds(j * L, L)] = zv
    for k in range(NPAD // NS // C):
        pltpu.sync_copy(msg0, acc.at[pl.ds((sid * 8 + k) * C, C)])
    plsc.subcore_barrier()

    issue_gathers(0, 0)
    issue_gathers(1, 1)

    def pair(it, _):
        for b in range(2):
            c = 2 * it + b
            wait_gathers(b)
            compute(c, b)

            @pl.when(c < NCHUNKS - 2)
            def _():
                issue_gathers(c + 2, b)
        return 0
    lax.fori_loop(0, (NCHUNKS - 1) // 2, pair, 0)

    wait_gathers(0)
    compute(NCHUNKS - 1, 0)
    pltpu.make_async_copy(msg1, acc.at[didx_all.at[0]], ss1).wait()
    pltpu.make_async_copy(msg0, acc.at[didx_all.at[0]], ss0).wait()

    plsc.subcore_barrier()
    rpw = NPAD // NS
    pltpu.sync_copy(acc.at[pl.ds(sid * rpw, rpw)],
                    out_h.at[cid, pl.ds(sid * rpw, rpw)])


_sc2 = pl.kernel(
    _sc2_body,
    out_type=jax.ShapeDtypeStruct((NC, NPAD, 48), jnp.float32),
    mesh=_MESH,
    compiler_params=_SC_PARAMS,
    scratch_types=[
        pltpu.VMEM_SHARED((NPAD, 48), jnp.float32),
        pltpu.VMEM((NCHUNKS, C), jnp.int32),
        pltpu.VMEM((NCHUNKS, C), jnp.int32),
        pltpu.VMEM((C, 32), jnp.float32),
        pltpu.VMEM((C, 32), jnp.float32),
        pltpu.VMEM((C, 16), jnp.float32),
        pltpu.VMEM((C, 16), jnp.float32),
        pltpu.VMEM((C, 16), jnp.float32),
        pltpu.VMEM((C, 16), jnp.float32),
        pltpu.VMEM((C, 48), jnp.float32),
        pltpu.VMEM((C, 48), jnp.float32),
        pltpu.VMEM((C,), jnp.float32),
        pltpu.SemaphoreType.DMA,
        pltpu.SemaphoreType.DMA,
        pltpu.SemaphoreType.DMA,
        pltpu.SemaphoreType.DMA,
    ],
)


def _sc3_body(zsd_h, zdd_h, src2_h, dst2_h, out_h,
              sidx_all, didx_all, ga0, ga1, gb0, gb1, zout0, zout1,
              sg0, sg1, sw0, sw1):
    cid = lax.axis_index("c")
    sid = lax.axis_index("s")
    wid = cid * NS + sid
    ga = (ga0, ga1)
    gb = (gb0, gb1)
    zout = (zout0, zout1)
    sg = (sg0, sg1)
    sw = (sw0, sw1)
    ebase = wid * EPW

    cbase = wid * NCHUNKS
    pltpu.sync_copy(src2_h.at[pl.ds(cbase, NCHUNKS)], sidx_all)
    pltpu.sync_copy(dst2_h.at[pl.ds(cbase, NCHUNKS)], didx_all)

    def issue_gathers(c, b):
        pltpu.async_copy(zsd_h.at[sidx_all.at[c]], ga[b], sg[b])
        pltpu.async_copy(zdd_h.at[didx_all.at[c]], gb[b], sg[b])

    def wait_gathers(b):
        pltpu.make_async_copy(zsd_h.at[sidx_all.at[0]], ga[b], sg[b]).wait()
        pltpu.make_async_copy(zdd_h.at[didx_all.at[0]], gb[b], sg[b]).wait()

    def compute(c, b):
        base = ebase + c * C

        @pl.when(c >= 2)
        def _():
            pltpu.make_async_copy(zout[b], out_h.at[pl.ds(base, C)],
                                  sw[b]).wait()

        @plsc.parallel_loop(0, C, 1, unroll=4)
        def _edge(e):
            zout[b][e, pl.ds(0, L)] = (ga[b][e, pl.ds(0, L)]
                                       + gb[b][e, pl.ds(0, L)])

        pltpu.async_copy(zout[b], out_h.at[pl.ds(base, C)], sw[b])

    issue_gathers(0, 0)
    issue_gathers(1, 1)

    def pair(it, _):
        for b in range(2):
            c = 2 * it + b
            wait_gathers(b)
            compute(c, b)

            @pl.when(c < NCHUNKS - 2)
            def _():
                issue_gathers(c + 2, b)
        return 0
    lax.fori_loop(0, (NCHUNKS - 1) // 2, pair, 0)

    wait_gathers(0)
    compute(NCHUNKS - 1, 0)
    pltpu.make_async_copy(zout1, out_h.at[pl.ds(ebase, C)], sw1).wait()
    pltpu.make_async_copy(zout0, out_h.at[pl.ds(ebase, C)], sw0).wait()


_sc3 = pl.kernel(
    _sc3_body,
    out_type=jax.ShapeDtypeStruct((E, 16), jnp.float32),
    mesh=_MESH,
    compiler_params=_SC_PARAMS,
    scratch_types=[
        pltpu.VMEM((NCHUNKS, C), jnp.int32),
        pltpu.VMEM((NCHUNKS, C), jnp.int32),
        pltpu.VMEM((C, 16), jnp.float32),
        pltpu.VMEM((C, 16), jnp.float32),
        pltpu.VMEM((C, 16), jnp.float32),
        pltpu.VMEM((C, 16), jnp.float32),
        pltpu.VMEM((C, 16), jnp.float32),
        pltpu.VMEM((C, 16), jnp.float32),
        pltpu.SemaphoreType.DMA,
        pltpu.SemaphoreType.DMA,
        pltpu.SemaphoreType.DMA,
        pltpu.SemaphoreType.DMA,
    ],
)


def kernel(n_feats, edge_index, edge_attr, W1, attn_l1, attn_r1, b1,
           W2, attn_l2, attn_r2, b2, Wp1, bp1, gamma, beta, Wp2, bp2):
    src = edge_index[0].reshape(E // C, C)
    dst = edge_index[1].reshape(E // C, C)
    src1 = edge_index[0].reshape(E // C1, C1)
    dst1 = edge_index[1].reshape(E // C1, C1)
    f32 = jnp.float32

    alT = (jnp.eye(8, dtype=f32)[:, None, :]
           * attn_l1[:, :, None]).reshape(128, 8)
    arT = (jnp.eye(8, dtype=f32)[:, None, :]
           * attn_r1[:, :, None]).reshape(128, 8)
    ex8 = jnp.repeat(jnp.eye(8, dtype=f32), 16, axis=1)
    a2t = jnp.concatenate([attn_l2.reshape(32, 1),
                           attn_r2.reshape(32, 1)], axis=1)
    wa = Wp1[:, :16]
    wsdT = jnp.concatenate([Wp1[:, 16:48].T, Wp1[:, 48:80].T], axis=1)
    waT16 = jnp.concatenate([wa.T, jnp.zeros((16, 8), f32)], axis=1)
    bp1_16 = jnp.concatenate([bp1, jnp.zeros((8,), f32)]).reshape(1, 16)
    g16 = jnp.concatenate([gamma, jnp.ones((8,), f32)]).reshape(1, 16)
    bt16 = jnp.concatenate([beta, jnp.zeros((8,), f32)]).reshape(1, 16)
    w2T = jnp.concatenate([Wp2.T, jnp.zeros((8, 2), f32)], axis=0)
    bp2_r = bp2.reshape(1, 2)

    h, elt, ert = _tc1(n_feats, W1.T, alT, arT)
    out1 = _sc1(h, elt, ert, src1, dst1)
    ht2, elt2, ert2 = _tc2(out1, b1.reshape(1, 128), ex8, W2.T, a2t)

    out2 = _sc2(ht2, elt2, ert2, src, dst)
    zsd, zdd = _tc3(out2, b2.reshape(1, 32), wsdT)

    zp = _sc3(zsd, zdd, src, dst)
    za, ps = _tc4a(edge_attr, zp, waT16, bp1_16)
    return _tc4b(za, ps, g16, bt16, w2T, bp2_r)


# speedup vs baseline: 51.9112x; 1.0481x over previous
import jax
import jax.numpy as jnp
from jax import lax
from jax.experimental import pallas as pl
from jax.experimental.pallas import tpu as pltpu
from jax.experimental.pallas import tpu_sc as plsc

N = 10000
E = 320000
NC, NS, L = 2, 16, 16
NW = NC * NS
EPW = E // NW
C = 80
NCHUNKS = EPW // C
NPAD = 10240
BN = 5000
BE = 16000
NBE = E // BE

_SC_PARAMS = pltpu.CompilerParams(use_tc_tiling_on_sc=False,
                                  needs_layout_passes=False)
_MESH = plsc.VectorSubcoreMesh(core_axis_name="c", subcore_axis_name="s",
                               num_cores=NC, num_subcores=NS)


def _tc1_body(x_ref, w_ref, alT_ref, arT_ref, h_ref, elt_ref, ert_ref):
    h = jnp.dot(x_ref[...], w_ref[...], preferred_element_type=jnp.float32)
    h_ref[...] = h
    el = jnp.dot(h, alT_ref[...], preferred_element_type=jnp.float32)
    er = jnp.dot(h, arT_ref[...], preferred_element_type=jnp.float32)
    z = jnp.zeros_like(el)
    elt_ref[...] = jnp.concatenate([el, z], axis=1)
    ert_ref[...] = jnp.concatenate([er, z], axis=1)


def _tc1(x, w1t, alT, arT):
    return pl.pallas_call(
        _tc1_body,
        out_shape=[jax.ShapeDtypeStruct((N, 128), jnp.float32),
                   jax.ShapeDtypeStruct((N, 16), jnp.float32),
                   jax.ShapeDtypeStruct((N, 16), jnp.float32)],
        grid=(N // BN,),
        in_specs=[pl.BlockSpec((BN, 128), lambda i: (i, 0)),
                  pl.BlockSpec((128, 128), lambda i: (0, 0)),
                  pl.BlockSpec((128, 8), lambda i: (0, 0)),
                  pl.BlockSpec((128, 8), lambda i: (0, 0))],
        out_specs=[pl.BlockSpec((BN, 128), lambda i: (i, 0)),
                   pl.BlockSpec((BN, 16), lambda i: (i, 0)),
                   pl.BlockSpec((BN, 16), lambda i: (i, 0))],
    )(x, w1t, alT, arT)


def _tc2_body(o_ref, b1_ref, ex8_ref, w2t_ref, a2t_ref,
              ht2_ref, elt2_ref, ert2_ref):
    agg = o_ref[0, :, :128] + o_ref[1, :, :128]
    es = o_ref[0, :, 128:136] + o_ref[1, :, 128:136]
    es128 = jnp.dot(es, ex8_ref[...], preferred_element_type=jnp.float32)
    h1 = jnp.maximum(agg / (es128 + 1e-9) + b1_ref[...], 0.0)
    h2p = jnp.dot(h1, w2t_ref[...], preferred_element_type=jnp.float32)
    ea2 = jnp.dot(h2p, a2t_ref[...], preferred_element_type=jnp.float32)
    ht2_ref[...] = h2p
    z15 = jnp.zeros((h2p.shape[0], 15), jnp.float32)
    elt2_ref[...] = jnp.concatenate([ea2[:, 0:1], z15], axis=1)
    ert2_ref[...] = jnp.concatenate([ea2[:, 1:2], z15], axis=1)


def _tc2(out1, b1, ex8, w2t, a2t):
    return pl.pallas_call(
        _tc2_body,
        out_shape=[jax.ShapeDtypeStruct((N, 32), jnp.float32),
                   jax.ShapeDtypeStruct((N, 16), jnp.float32),
                   jax.ShapeDtypeStruct((N, 16), jnp.float32)],
        grid=(N // BN,),
        in_specs=[pl.BlockSpec((NC, BN, 144), lambda i: (0, i, 0)),
                  pl.BlockSpec((1, 128), lambda i: (0, 0)),
                  pl.BlockSpec((8, 128), lambda i: (0, 0)),
                  pl.BlockSpec((128, 32), lambda i: (0, 0)),
                  pl.BlockSpec((32, 2), lambda i: (0, 0))],
        out_specs=[pl.BlockSpec((BN, 32), lambda i: (i, 0)),
                   pl.BlockSpec((BN, 16), lambda i: (i, 0)),
                   pl.BlockSpec((BN, 16), lambda i: (i, 0))],
    )(out1, b1, ex8, w2t, a2t)


def _tc3_body(o_ref, b2_ref, wsdT_ref, zsd_ref, zdd_ref):
    agg = o_ref[0, :, :32] + o_ref[1, :, :32]
    es = o_ref[0, :, 32:33] + o_ref[1, :, 32:33]
    h2 = jnp.maximum(agg / (es + 1e-9) + b2_ref[...], 0.0)
    zz = jnp.dot(h2, wsdT_ref[...], preferred_element_type=jnp.float32)
    zsd_ref[...] = zz
    zdd_ref[...] = jnp.concatenate([zz[:, 8:], zz[:, 8:]], axis=1)


def _tc3(out2, b2, wsdT):
    return pl.pallas_call(
        _tc3_body,
        out_shape=[jax.ShapeDtypeStruct((N, 16), jnp.float32),
                   jax.ShapeDtypeStruct((N, 16), jnp.float32)],
        grid=(N // BN,),
        in_specs=[pl.BlockSpec((NC, BN, 48), lambda i: (0, i, 0)),
                  pl.BlockSpec((1, 32), lambda i: (0, 0)),
                  pl.BlockSpec((32, 16), lambda i: (0, 0))],
        out_specs=[pl.BlockSpec((BN, 16), lambda i: (i, 0)),
                   pl.BlockSpec((BN, 16), lambda i: (i, 0))],
    )(out2, b2, wsdT)


def _tc4a_body(ea_ref, zp_ref, waT_ref, bp1_ref, za_ref, ps_ref):
    za = (zp_ref[...]
          + jnp.dot(ea_ref[...], waT_ref[...],
                    preferred_element_type=jnp.float32)
          + bp1_ref[...])
    za_ref[...] = za
    ps_ref[0, 0, :] = jnp.sum(za, axis=0)
    ps_ref[0, 1, :] = jnp.sum(za * za, axis=0)


def _tc4a(ea, zp, waT, bp1_16):
    return pl.pallas_call(
        _tc4a_body,
        out_shape=[jax.ShapeDtypeStruct((E, 16), jnp.float32),
                   jax.ShapeDtypeStruct((NBE, 2, 16), jnp.float32)],
        grid=(NBE,),
        in_specs=[pl.BlockSpec((BE, 16), lambda i: (i, 0)),
                  pl.BlockSpec((BE, 16), lambda i: (i, 0)),
                  pl.BlockSpec((16, 16), lambda i: (0, 0)),
                  pl.BlockSpec((1, 16), lambda i: (0, 0))],
        out_specs=[pl.BlockSpec((BE, 16), lambda i: (i, 0)),
                   pl.BlockSpec((1, 2, 16), lambda i: (i, 0, 0))],
    )(ea, zp, waT, bp1_16)


def _tc4b_body(za_ref, ps_ref, g_ref, bt_ref, w2T_ref, bp2_ref, out_ref):
    tot = jnp.sum(ps_ref[...], axis=0)
    mean = tot[0] * (1.0 / E)
    var = tot[1] * (1.0 / E) - mean * mean
    zn = ((za_ref[...] - mean) / jnp.sqrt(var + 1e-5)
          * g_ref[...] + bt_ref[...])
    logits = (jnp.dot(zn, w2T_ref[...], preferred_element_type=jnp.float32)
              + bp2_ref[...])
    m = jnp.max(logits, axis=1, keepdims=True)
    s = logits - m
    out_ref[...] = s - jnp.log(jnp.sum(jnp.exp(s), axis=1, keepdims=True))


def _tc4b(za, ps, g16, bt16, w2T, bp2):
    return pl.pallas_call(
        _tc4b_body,
        out_shape=jax.ShapeDtypeStruct((E, 2), jnp.float32),
        grid=(NBE,),
        in_specs=[pl.BlockSpec((BE, 16), lambda i: (i, 0)),
                  pl.BlockSpec((NBE, 2, 16), lambda i: (0, 0, 0)),
                  pl.BlockSpec((1, 16), lambda i: (0, 0)),
                  pl.BlockSpec((1, 16), lambda i: (0, 0)),
                  pl.BlockSpec((16, 2), lambda i: (0, 0)),
                  pl.BlockSpec((1, 2), lambda i: (0, 0))],
        out_specs=pl.BlockSpec((BE, 2), lambda i: (i, 0)),
    )(za, ps, g16, bt16, w2T, bp2)


C1 = 40
NCH1 = EPW // C1
SEG = 50
NSEG = NCH1 // SEG


def _sc1_body(ht_h, elt_h, ert_h, src2_h, dst2_h, out_h,
              acc, sslab, dslab, rows0, rows1, elr0, elr1,
              err0, err1, msg0, msg1, exb, sg0, sg1, ss0, ss1):
    cid = lax.axis_index("c")
    sid = lax.axis_index("s")
    wid = cid * NS + sid
    zv = jnp.zeros((L,), jnp.float32)
    iota = lax.iota(jnp.int32, L)
    rows = (rows0, rows1)
    elr = (elr0, elr1)
    err = (err0, err1)
    msg = (msg0, msg1)
    sg = (sg0, sg1)
    ss = (ss0, ss1)

    def issue_gathers(c, b):
        pltpu.async_copy(ht_h.at[sslab.at[c]], rows[b], sg[b])
        pltpu.async_copy(elt_h.at[sslab.at[c]], elr[b].at[pl.ds(0, C1)], sg[b])
        pltpu.async_copy(ert_h.at[dslab.at[c]], err[b].at[pl.ds(0, C1)], sg[b])

    def wait_gathers(b):
        pltpu.make_async_copy(ht_h.at[sslab.at[0]], rows[b], sg[b]).wait()
        pltpu.make_async_copy(elt_h.at[sslab.at[0]],
                              elr[b].at[pl.ds(0, C1)], sg[b]).wait()
        pltpu.make_async_copy(ert_h.at[dslab.at[0]],
                              err[b].at[pl.ds(0, C1)], sg[b]).wait()

    def drain_scatter(b):
        pltpu.make_async_copy(msg[b], acc.at[dslab.at[0]], ss[b]).wait()

    def compute(c, b):
        def grp(g, _):
            rid = g * L + iota
            for h in range(8):
                hf = jnp.full((L,), h, jnp.int32)
                e = (plsc.load_gather(elr[b], [rid, hf])
                     + plsc.load_gather(err[b], [rid, hf]))
                e = jnp.maximum(e, 0.2 * e)
                exb[h, pl.ds(g * L, L)] = jnp.exp(e)
            return 0
        lax.fori_loop(0, 3, grp, 0)

        @pl.when(c >= 2)
        def _():
            drain_scatter(b)

        @plsc.parallel_loop(0, C1, 1, unroll=2)
        def _edge(e):
            ef = jnp.full((L,), e, jnp.int32)
            wv = plsc.load_gather(exb, [iota % 8, ef])
            msg[b][e, pl.ds(128, L)] = wv
            for j in range(8):
                w = wv[jnp.full((L,), j, jnp.int32)]
                msg[b][e, pl.ds(j * L, L)] = w * rows[b][e, pl.ds(j * L, L)]

        pltpu.async_copy(msg[b], acc.at[dslab.at[c]], ss[b], add=True)

    @plsc.parallel_loop(0, C1, 1, unroll=4)
    def _zrow(r):
        for j in range(9):
            msg0[r, pl.ds(j * L, L)] = zv
    for k in range(NPAD // NS // C1):
        pltpu.sync_copy(msg0, acc.at[pl.ds((sid * 16 + k) * C1, C1)])
    plsc.subcore_barrier()

    def segment(s, _):
        @pl.when(s > 0)
        def _():
            drain_scatter(0)
            drain_scatter(1)
        cbase = wid * NCH1 + s * SEG
        pltpu.sync_copy(src2_h.at[pl.ds(cbase, SEG)], sslab)
        pltpu.sync_copy(dst2_h.at[pl.ds(cbase, SEG)], dslab)
        issue_gathers(0, 0)
        issue_gathers(1, 1)

        def pair(it, _):
            for b in range(2):
                c = 2 * it + b
                wait_gathers(b)
                compute(c, b)

                @pl.when(c < SEG - 2)
                def _():
                    issue_gathers(c + 2, b)
            return 0
        lax.fori_loop(0, SEG // 2, pair, 0)
        return 0
    lax.fori_loop(0, NSEG, segment, 0)

    drain_scatter(0)
    drain_scatter(1)
    plsc.subcore_barrier()
    rpw = NPAD // NS
    pltpu.sync_copy(acc.at[pl.ds(sid * rpw, rpw)],
                    out_h.at[cid, pl.ds(sid * rpw, rpw)])


_sc1 = pl.kernel(
    _sc1_body,
    out_type=jax.ShapeDtypeStruct((NC, NPAD, 144), jnp.float32),
    mesh=_MESH,
    compiler_params=_SC_PARAMS,
    scratch_types=[
        pltpu.VMEM_SHARED((NPAD, 144), jnp.float32),
        pltpu.VMEM((SEG, C1), jnp.int32),
        pltpu.VMEM((SEG, C1), jnp.int32),
        pltpu.VMEM((C1, 128), jnp.float32),
        pltpu.VMEM((C1, 128), jnp.float32),
        pltpu.VMEM((48, 16), jnp.float32),
        pltpu.VMEM((48, 16), jnp.float32),
        pltpu.VMEM((48, 16), jnp.float32),
        pltpu.VMEM((48, 16), jnp.float32),
        pltpu.VMEM((C1, 144), jnp.float32),
        pltpu.VMEM((C1, 144), jnp.float32),
        pltpu.VMEM((8, 48), jnp.float32),
        pltpu.SemaphoreType.DMA,
        pltpu.SemaphoreType.DMA,
        pltpu.SemaphoreType.DMA,
        pltpu.SemaphoreType.DMA,
    ],
)


def _sc2_body(ht_h, elt_h, ert_h, src2_h, dst2_h, out_h,
              acc, sidx_all, didx_all, rows0, rows1, elr0, elr1,
              err0, err1, msg0, msg1, exv, sg0, sg1, ss0, ss1):
    cid = lax.axis_index("c")
    sid = lax.axis_index("s")
    wid = cid * NS + sid
    zv = jnp.zeros((L,), jnp.float32)
    iota = lax.iota(jnp.int32, L)
    zf = jnp.zeros((L,), jnp.int32)
    rows = (rows0, rows1)
    elr = (elr0, elr1)
    err = (err0, err1)
    msg = (msg0, msg1)
    sg = (sg0, sg1)
    ss = (ss0, ss1)

    cbase = wid * NCHUNKS
    pltpu.sync_copy(src2_h.at[pl.ds(cbase, NCHUNKS)], sidx_all)
    pltpu.sync_copy(dst2_h.at[pl.ds(cbase, NCHUNKS)], didx_all)

    def issue_gathers(c, b):
        pltpu.async_copy(ht_h.at[sidx_all.at[c]], rows[b], sg[b])
        pltpu.async_copy(elt_h.at[sidx_all.at[c]], elr[b], sg[b])
        pltpu.async_copy(ert_h.at[didx_all.at[c]], err[b], sg[b])

    def wait_gathers(b):
        pltpu.make_async_copy(ht_h.at[sidx_all.at[0]], rows[b], sg[b]).wait()
        pltpu.make_async_copy(elt_h.at[sidx_all.at[0]], elr[b], sg[b]).wait()
        pltpu.make_async_copy(ert_h.at[didx_all.at[0]], err[b], sg[b]).wait()

    def compute(c, b):
        def grp(g, _):
            rid = g * L + iota
            e = (plsc.load_gather(elr[b], [rid, zf])
                 + plsc.load_gather(err[b], [rid, zf]))
            e = jnp.maximum(e, 0.2 * e)
            exv[pl.ds(g * L, L)] = jnp.exp(e)
            return 0
        lax.fori_loop(0, C // L, grp, 0)

        @pl.when(c >= 2)
        def _():
            pltpu.make_async_copy(msg[b], acc.at[didx_all.at[c]],
                                  ss[b]).wait()

        @plsc.parallel_loop(0, C, 1, unroll=4)
        def _edge(e):
            w = plsc.load_gather(exv, [jnp.full((L,), e, jnp.int32)])
            msg[b][e, pl.ds(0, L)] = w * rows[b][e, pl.ds(0, L)]
            msg[b][e, pl.ds(L, L)] = w * rows[b][e, pl.ds(L, L)]
            msg[b][e, pl.ds(2 * L, L)] = w

        pltpu.async_copy(msg[b], acc.at[didx_all.at[c]], ss[b], add=True)

    @plsc.parallel_loop(0, C, 1, unroll=4)
    def _zrow(r):
        for j in range(3):
            msg0[r, pl.ds(j * L, L)] = zv
    for k in range(NPAD // NS // C):
        pltpu.sync_copy(msg0, acc.at[pl.ds((sid * 8 + k) * C, C)])
    plsc.subcore_barrier()

    issue_gathers(0, 0)
    issue_gathers(1, 1)

    def pair(it, _):
        for b in range(2):
            c = 2 * it + b
            wait_gathers(b)
            compute(c, b)

            @pl.when(c < NCHUNKS - 2)
            def _():
                issue_gathers(c + 2, b)
        return 0
    lax.fori_loop(0, (NCHUNKS - 1) // 2, pair, 0)

    wait_gathers(0)
    compute(NCHUNKS - 1, 0)
    pltpu.make_async_copy(msg1, acc.at[didx_all.at[0]], ss1).wait()
    pltpu.make_async_copy(msg0, acc.at[didx_all.at[0]], ss0).wait()

    plsc.subcore_barrier()
    rpw = NPAD // NS
    pltpu.sync_copy(acc.at[pl.ds(sid * rpw, rpw)],
                    out_h.at[cid, pl.ds(sid * rpw, rpw)])


_sc2 = pl.kernel(
    _sc2_body,
    out_type=jax.ShapeDtypeStruct((NC, NPAD, 48), jnp.float32),
    mesh=_MESH,
    compiler_params=_SC_PARAMS,
    scratch_types=[
        pltpu.VMEM_SHARED((NPAD, 48), jnp.float32),
        pltpu.VMEM((NCHUNKS, C), jnp.int32),
        pltpu.VMEM((NCHUNKS, C), jnp.int32),
        pltpu.VMEM((C, 32), jnp.float32),
        pltpu.VMEM((C, 32), jnp.float32),
        pltpu.VMEM((C, 16), jnp.float32),
        pltpu.VMEM((C, 16), jnp.float32),
        pltpu.VMEM((C, 16), jnp.float32),
        pltpu.VMEM((C, 16), jnp.float32),
        pltpu.VMEM((C, 48), jnp.float32),
        pltpu.VMEM((C, 48), jnp.float32),
        pltpu.VMEM((C,), jnp.float32),
        pltpu.SemaphoreType.DMA,
        pltpu.SemaphoreType.DMA,
        pltpu.SemaphoreType.DMA,
        pltpu.SemaphoreType.DMA,
    ],
)


def _sc3_body(zsd_h, zdd_h, src2_h, dst2_h, out_h,
              sidx_all, didx_all, ga0, ga1, gb0, gb1, zout0, zout1,
              sg0, sg1, sw0, sw1):
    cid = lax.axis_index("c")
    sid = lax.axis_index("s")
    wid = cid * NS + sid
    ga = (ga0, ga1)
    gb = (gb0, gb1)
    zout = (zout0, zout1)
    sg = (sg0, sg1)
    sw = (sw0, sw1)
    ebase = wid * EPW

    cbase = wid * NCHUNKS
    pltpu.sync_copy(src2_h.at[pl.ds(cbase, NCHUNKS)], sidx_all)
    pltpu.sync_copy(dst2_h.at[pl.ds(cbase, NCHUNKS)], didx_all)

    def issue_gathers(c, b):
        pltpu.async_copy(zsd_h.at[sidx_all.at[c]], ga[b], sg[b])
        pltpu.async_copy(zdd_h.at[didx_all.at[c]], gb[b], sg[b])

    def wait_gathers(b):
        pltpu.make_async_copy(zsd_h.at[sidx_all.at[0]], ga[b], sg[b]).wait()
        pltpu.make_async_copy(zdd_h.at[didx_all.at[0]], gb[b], sg[b]).wait()

    def compute(c, b):
        base = ebase + c * C

        @pl.when(c >= 2)
        def _():
            pltpu.make_async_copy(zout[b], out_h.at[pl.ds(base, C)],
                                  sw[b]).wait()

        @plsc.parallel_loop(0, C, 1, unroll=4)
        def _edge(e):
            zout[b][e, pl.ds(0, L)] = (ga[b][e, pl.ds(0, L)]
                                       + gb[b][e, pl.ds(0, L)])

        pltpu.async_copy(zout[b], out_h.at[pl.ds(base, C)], sw[b])

    issue_gathers(0, 0)
    issue_gathers(1, 1)

    def pair(it, _):
        for b in range(2):
            c = 2 * it + b
            wait_gathers(b)
            compute(c, b)

            @pl.when(c < NCHUNKS - 2)
            def _():
                issue_gathers(c + 2, b)
        return 0
    lax.fori_loop(0, (NCHUNKS - 1) // 2, pair, 0)

    wait_gathers(0)
    compute(NCHUNKS - 1, 0)
    pltpu.make_async_copy(zout1, out_h.at[pl.ds(ebase, C)], sw1).wait()
    pltpu.make_async_copy(zout0, out_h.at[pl.ds(ebase, C)], sw0).wait()


_sc3 = pl.kernel(
    _sc3_body,
    out_type=jax.ShapeDtypeStruct((E, 16), jnp.float32),
    mesh=_MESH,
    compiler_params=_SC_PARAMS,
    scratch_types=[
        pltpu.VMEM((NCHUNKS, C), jnp.int32),
        pltpu.VMEM((NCHUNKS, C), jnp.int32),
        pltpu.VMEM((C, 16), jnp.float32),
        pltpu.VMEM((C, 16), jnp.float32),
        pltpu.VMEM((C, 16), jnp.float32),
        pltpu.VMEM((C, 16), jnp.float32),
        pltpu.VMEM((C, 16), jnp.float32),
        pltpu.VMEM((C, 16), jnp.float32),
        pltpu.SemaphoreType.DMA,
        pltpu.SemaphoreType.DMA,
        pltpu.SemaphoreType.DMA,
        pltpu.SemaphoreType.DMA,
    ],
)


def kernel(n_feats, edge_index, edge_attr, W1, attn_l1, attn_r1, b1,
           W2, attn_l2, attn_r2, b2, Wp1, bp1, gamma, beta, Wp2, bp2):
    src = edge_index[0].reshape(E // C, C)
    dst = edge_index[1].reshape(E // C, C)
    src1 = edge_index[0].reshape(E // C1, C1)
    dst1 = edge_index[1].reshape(E // C1, C1)
    f32 = jnp.float32

    alT = (jnp.eye(8, dtype=f32)[:, None, :]
           * attn_l1[:, :, None]).reshape(128, 8)
    arT = (jnp.eye(8, dtype=f32)[:, None, :]
           * attn_r1[:, :, None]).reshape(128, 8)
    ex8 = jnp.repeat(jnp.eye(8, dtype=f32), 16, axis=1)
    a2t = jnp.concatenate([attn_l2.reshape(32, 1),
                           attn_r2.reshape(32, 1)], axis=1)
    wa = Wp1[:, :16]
    wsdT = jnp.concatenate([Wp1[:, 16:48].T, Wp1[:, 48:80].T], axis=1)
    waT16 = jnp.concatenate([wa.T, jnp.zeros((16, 8), f32)], axis=1)
    bp1_16 = jnp.concatenate([bp1, jnp.zeros((8,), f32)]).reshape(1, 16)
    g16 = jnp.concatenate([gamma, jnp.ones((8,), f32)]).reshape(1, 16)
    bt16 = jnp.concatenate([beta, jnp.zeros((8,), f32)]).reshape(1, 16)
    w2T = jnp.concatenate([Wp2.T, jnp.zeros((8, 2), f32)], axis=0)
    bp2_r = bp2.reshape(1, 2)

    h, elt, ert = _tc1(n_feats, W1.T, alT, arT)
    out1 = _sc1(h, elt, ert, src1, dst1)
    ht2, elt2, ert2 = _tc2(out1, b1.reshape(1, 128), ex8, W2.T, a2t)

    out2 = _sc2(ht2, elt2, ert2, src, dst)
    zsd, zdd = _tc3(out2, b2.reshape(1, 32), wsdT)

    zp = _sc3(zsd, zdd, src, dst)
    za, ps = _tc4a(edge_attr, zp, waT16, bp1_16)
    return _tc4b(za, ps, g16, bt16, w2T, bp2_r)


# speedup vs baseline: 52.1307x; 1.0042x over previous
import jax
import jax.numpy as jnp
from jax import lax
from jax.experimental import pallas as pl
from jax.experimental.pallas import tpu as pltpu
from jax.experimental.pallas import tpu_sc as plsc

N = 10000
E = 320000
NC, NS, L = 2, 16, 16
NW = NC * NS
EPW = E // NW
C = 80
NCHUNKS = EPW // C
NPAD = 10240
BN = 5000
BE = 16000
NBE = E // BE

_SC_PARAMS = pltpu.CompilerParams(use_tc_tiling_on_sc=False,
                                  needs_layout_passes=False)
_MESH = plsc.VectorSubcoreMesh(core_axis_name="c", subcore_axis_name="s",
                               num_cores=NC, num_subcores=NS)


def _tc1_body(x_ref, w_ref, alT_ref, arT_ref, h_ref, elt_ref, ert_ref):
    h = jnp.dot(x_ref[...], w_ref[...], preferred_element_type=jnp.float32)
    h_ref[...] = h
    el = jnp.dot(h, alT_ref[...], preferred_element_type=jnp.float32)
    er = jnp.dot(h, arT_ref[...], preferred_element_type=jnp.float32)
    z = jnp.zeros_like(el)
    elt_ref[...] = jnp.concatenate([el, z], axis=1)
    ert_ref[...] = jnp.concatenate([er, z], axis=1)


def _tc1(x, w1t, alT, arT):
    return pl.pallas_call(
        _tc1_body,
        out_shape=[jax.ShapeDtypeStruct((N, 128), jnp.float32),
                   jax.ShapeDtypeStruct((N, 16), jnp.float32),
                   jax.ShapeDtypeStruct((N, 16), jnp.float32)],
        grid=(N // BN,),
        in_specs=[pl.BlockSpec((BN, 128), lambda i: (i, 0)),
                  pl.BlockSpec((128, 128), lambda i: (0, 0)),
                  pl.BlockSpec((128, 8), lambda i: (0, 0)),
                  pl.BlockSpec((128, 8), lambda i: (0, 0))],
        out_specs=[pl.BlockSpec((BN, 128), lambda i: (i, 0)),
                   pl.BlockSpec((BN, 16), lambda i: (i, 0)),
                   pl.BlockSpec((BN, 16), lambda i: (i, 0))],
    )(x, w1t, alT, arT)


def _tc2_body(o_ref, b1_ref, ex8_ref, w2t_ref, a2t_ref,
              ht2_ref, elt2_ref, ert2_ref):
    agg = o_ref[0, :, :128] + o_ref[1, :, :128]
    es = o_ref[0, :, 128:136] + o_ref[1, :, 128:136]
    es128 = jnp.dot(es, ex8_ref[...], preferred_element_type=jnp.float32)
    h1 = jnp.maximum(agg / (es128 + 1e-9) + b1_ref[...], 0.0)
    h2p = jnp.dot(h1, w2t_ref[...], preferred_element_type=jnp.float32)
    ea2 = jnp.dot(h2p, a2t_ref[...], preferred_element_type=jnp.float32)
    ht2_ref[...] = h2p
    z15 = jnp.zeros((h2p.shape[0], 15), jnp.float32)
    elt2_ref[...] = jnp.concatenate([ea2[:, 0:1], z15], axis=1)
    ert2_ref[...] = jnp.concatenate([ea2[:, 1:2], z15], axis=1)


def _tc2(out1, b1, ex8, w2t, a2t):
    return pl.pallas_call(
        _tc2_body,
        out_shape=[jax.ShapeDtypeStruct((N, 32), jnp.float32),
                   jax.ShapeDtypeStruct((N, 16), jnp.float32),
                   jax.ShapeDtypeStruct((N, 16), jnp.float32)],
        grid=(N // BN,),
        in_specs=[pl.BlockSpec((NC, BN, 144), lambda i: (0, i, 0)),
                  pl.BlockSpec((1, 128), lambda i: (0, 0)),
                  pl.BlockSpec((8, 128), lambda i: (0, 0)),
                  pl.BlockSpec((128, 32), lambda i: (0, 0)),
                  pl.BlockSpec((32, 2), lambda i: (0, 0))],
        out_specs=[pl.BlockSpec((BN, 32), lambda i: (i, 0)),
                   pl.BlockSpec((BN, 16), lambda i: (i, 0)),
                   pl.BlockSpec((BN, 16), lambda i: (i, 0))],
    )(out1, b1, ex8, w2t, a2t)


def _tc3_body(o_ref, b2_ref, wsdT_ref, zsd_ref, zdd_ref):
    agg = o_ref[0, :, :32] + o_ref[1, :, :32]
    es = o_ref[0, :, 32:33] + o_ref[1, :, 32:33]
    h2 = jnp.maximum(agg / (es + 1e-9) + b2_ref[...], 0.0)
    zz = jnp.dot(h2, wsdT_ref[...], preferred_element_type=jnp.float32)
    zsd_ref[...] = zz
    zdd_ref[...] = jnp.concatenate([zz[:, 8:], zz[:, 8:]], axis=1)


def _tc3(out2, b2, wsdT):
    return pl.pallas_call(
        _tc3_body,
        out_shape=[jax.ShapeDtypeStruct((N, 16), jnp.float32),
                   jax.ShapeDtypeStruct((N, 16), jnp.float32)],
        grid=(N // BN,),
        in_specs=[pl.BlockSpec((NC, BN, 48), lambda i: (0, i, 0)),
                  pl.BlockSpec((1, 32), lambda i: (0, 0)),
                  pl.BlockSpec((32, 16), lambda i: (0, 0))],
        out_specs=[pl.BlockSpec((BN, 16), lambda i: (i, 0)),
                   pl.BlockSpec((BN, 16), lambda i: (i, 0))],
    )(out2, b2, wsdT)


def _tc4a_body(ea_ref, zp_ref, waT_ref, bp1_ref, za_ref, ps_ref):
    za = (zp_ref[...]
          + jnp.dot(ea_ref[...], waT_ref[...],
                    preferred_element_type=jnp.float32)
          + bp1_ref[...])
    za_ref[...] = za
    ps_ref[0, 0, :] = jnp.sum(za, axis=0)
    ps_ref[0, 1, :] = jnp.sum(za * za, axis=0)


def _tc4a(ea, zp, waT, bp1_16):
    return pl.pallas_call(
        _tc4a_body,
        out_shape=[jax.ShapeDtypeStruct((E, 16), jnp.float32),
                   jax.ShapeDtypeStruct((NBE, 2, 16), jnp.float32)],
        grid=(NBE,),
        in_specs=[pl.BlockSpec((BE, 16), lambda i: (i, 0)),
                  pl.BlockSpec((BE, 16), lambda i: (i, 0)),
                  pl.BlockSpec((16, 16), lambda i: (0, 0)),
                  pl.BlockSpec((1, 16), lambda i: (0, 0))],
        out_specs=[pl.BlockSpec((BE, 16), lambda i: (i, 0)),
                   pl.BlockSpec((1, 2, 16), lambda i: (i, 0, 0))],
    )(ea, zp, waT, bp1_16)


def _tc4b_body(za_ref, ps_ref, g_ref, bt_ref, w2T_ref, bp2_ref, out_ref):
    tot = jnp.sum(ps_ref[...], axis=0)
    mean = tot[0] * (1.0 / E)
    var = tot[1] * (1.0 / E) - mean * mean
    zn = ((za_ref[...] - mean) / jnp.sqrt(var + 1e-5)
          * g_ref[...] + bt_ref[...])
    logits = (jnp.dot(zn, w2T_ref[...], preferred_element_type=jnp.float32)
              + bp2_ref[...])
    m = jnp.max(logits, axis=1, keepdims=True)
    s = logits - m
    out_ref[...] = s - jnp.log(jnp.sum(jnp.exp(s), axis=1, keepdims=True))


def _tc4b(za, ps, g16, bt16, w2T, bp2):
    return pl.pallas_call(
        _tc4b_body,
        out_shape=jax.ShapeDtypeStruct((E, 2), jnp.float32),
        grid=(NBE,),
        in_specs=[pl.BlockSpec((BE, 16), lambda i: (i, 0)),
                  pl.BlockSpec((NBE, 2, 16), lambda i: (0, 0, 0)),
                  pl.BlockSpec((1, 16), lambda i: (0, 0)),
                  pl.BlockSpec((1, 16), lambda i: (0, 0)),
                  pl.BlockSpec((16, 2), lambda i: (0, 0)),
                  pl.BlockSpec((1, 2), lambda i: (0, 0))],
        out_specs=pl.BlockSpec((BE, 2), lambda i: (i, 0)),
    )(za, ps, g16, bt16, w2T, bp2)


C1 = 40
NCH1 = EPW // C1
SEG = 50
NSEG = NCH1 // SEG


def _sc1_body(ht_h, elt_h, ert_h, src2_h, dst2_h, out_h,
              acc, sslab, dslab, rows0, rows1, elr0, elr1,
              err0, err1, msg0, msg1, exb, sg0, sg1, ss0, ss1):
    cid = lax.axis_index("c")
    sid = lax.axis_index("s")
    wid = cid * NS + sid
    zv = jnp.zeros((L,), jnp.float32)
    iota = lax.iota(jnp.int32, L)
    rows = (rows0, rows1)
    elr = (elr0, elr1)
    err = (err0, err1)
    msg = (msg0, msg1)
    sg = (sg0, sg1)
    ss = (ss0, ss1)

    def issue_gathers(c, b):
        pltpu.async_copy(ht_h.at[sslab.at[c]], rows[b], sg[b])
        pltpu.async_copy(elt_h.at[sslab.at[c]], elr[b].at[pl.ds(0, C1)], sg[b])
        pltpu.async_copy(ert_h.at[dslab.at[c]], err[b].at[pl.ds(0, C1)], sg[b])

    def wait_gathers(b):
        pltpu.make_async_copy(ht_h.at[sslab.at[0]], rows[b], sg[b]).wait()
        pltpu.make_async_copy(elt_h.at[sslab.at[0]],
                              elr[b].at[pl.ds(0, C1)], sg[b]).wait()
        pltpu.make_async_copy(ert_h.at[dslab.at[0]],
                              err[b].at[pl.ds(0, C1)], sg[b]).wait()

    def drain_scatter(b):
        pltpu.make_async_copy(msg[b], acc.at[dslab.at[0]], ss[b]).wait()

    def compute(c, b):
        def grp(g, _):
            rid = g * L + iota
            for h in range(8):
                hf = jnp.full((L,), h, jnp.int32)
                e = (plsc.load_gather(elr[b], [rid, hf])
                     + plsc.load_gather(err[b], [rid, hf]))
                e = jnp.maximum(e, 0.2 * e)
                exb[h, pl.ds(g * L, L)] = jnp.exp(e)
            return 0
        lax.fori_loop(0, 3, grp, 0)

        @pl.when(c >= 2)
        def _():
            drain_scatter(b)

        @plsc.parallel_loop(0, C1, 1, unroll=4)
        def _edge(e):
            ef = jnp.full((L,), e, jnp.int32)
            wv = plsc.load_gather(exb, [iota % 8, ef])
            msg[b][e, pl.ds(128, L)] = wv
            for j in range(8):
                w = wv[jnp.full((L,), j, jnp.int32)]
                msg[b][e, pl.ds(j * L, L)] = w * rows[b][e, pl.ds(j * L, L)]

        pltpu.async_copy(msg[b], acc.at[dslab.at[c]], ss[b], add=True)

    @plsc.parallel_loop(0, C1, 1, unroll=4)
    def _zrow(r):
        for j in range(9):
            msg0[r, pl.ds(j * L, L)] = zv
    for k in range(NPAD // NS // C1):
        pltpu.sync_copy(msg0, acc.at[pl.ds((sid * 16 + k) * C1, C1)])
    plsc.subcore_barrier()

    def segment(s, _):
        @pl.when(s > 0)
        def _():
            drain_scatter(0)
            drain_scatter(1)
        cbase = wid * NCH1 + s * SEG
        pltpu.sync_copy(src2_h.at[pl.ds(cbase, SEG)], sslab)
        pltpu.sync_copy(dst2_h.at[pl.ds(cbase, SEG)], dslab)
        issue_gathers(0, 0)
        issue_gathers(1, 1)

        def pair(it, _):
            for b in range(2):
                c = 2 * it + b
                wait_gathers(b)
                compute(c, b)

                @pl.when(c < SEG - 2)
                def _():
                    issue_gathers(c + 2, b)
            return 0
        lax.fori_loop(0, SEG // 2, pair, 0)
        return 0
    lax.fori_loop(0, NSEG, segment, 0)

    drain_scatter(0)
    drain_scatter(1)
    plsc.subcore_barrier()
    rpw = NPAD // NS
    pltpu.sync_copy(acc.at[pl.ds(sid * rpw, rpw)],
                    out_h.at[cid, pl.ds(sid * rpw, rpw)])


_sc1 = pl.kernel(
    _sc1_body,
    out_type=jax.ShapeDtypeStruct((NC, NPAD, 144), jnp.float32),
    mesh=_MESH,
    compiler_params=_SC_PARAMS,
    scratch_types=[
        pltpu.VMEM_SHARED((NPAD, 144), jnp.float32),
        pltpu.VMEM((SEG, C1), jnp.int32),
        pltpu.VMEM((SEG, C1), jnp.int32),
        pltpu.VMEM((C1, 128), jnp.float32),
        pltpu.VMEM((C1, 128), jnp.float32),
        pltpu.VMEM((48, 16), jnp.float32),
        pltpu.VMEM((48, 16), jnp.float32),
        pltpu.VMEM((48, 16), jnp.float32),
        pltpu.VMEM((48, 16), jnp.float32),
        pltpu.VMEM((C1, 144), jnp.float32),
        pltpu.VMEM((C1, 144), jnp.float32),
        pltpu.VMEM((8, 48), jnp.float32),
        pltpu.SemaphoreType.DMA,
        pltpu.SemaphoreType.DMA,
        pltpu.SemaphoreType.DMA,
        pltpu.SemaphoreType.DMA,
    ],
)


def _sc2_body(ht_h, elt_h, ert_h, src2_h, dst2_h, out_h,
              acc, sidx_all, didx_all, rows0, rows1, elr0, elr1,
              err0, err1, msg0, msg1, exv, sg0, sg1, ss0, ss1):
    cid = lax.axis_index("c")
    sid = lax.axis_index("s")
    wid = cid * NS + sid
    zv = jnp.zeros((L,), jnp.float32)
    iota = lax.iota(jnp.int32, L)
    zf = jnp.zeros((L,), jnp.int32)
    rows = (rows0, rows1)
    elr = (elr0, elr1)
    err = (err0, err1)
    msg = (msg0, msg1)
    sg = (sg0, sg1)
    ss = (ss0, ss1)

    cbase = wid * NCHUNKS
    pltpu.sync_copy(src2_h.at[pl.ds(cbase, NCHUNKS)], sidx_all)
    pltpu.sync_copy(dst2_h.at[pl.ds(cbase, NCHUNKS)], didx_all)

    def issue_gathers(c, b):
        pltpu.async_copy(ht_h.at[sidx_all.at[c]], rows[b], sg[b])
        pltpu.async_copy(elt_h.at[sidx_all.at[c]], elr[b], sg[b])
        pltpu.async_copy(ert_h.at[didx_all.at[c]], err[b], sg[b])

    def wait_gathers(b):
        pltpu.make_async_copy(ht_h.at[sidx_all.at[0]], rows[b], sg[b]).wait()
        pltpu.make_async_copy(elt_h.at[sidx_all.at[0]], elr[b], sg[b]).wait()
        pltpu.make_async_copy(ert_h.at[didx_all.at[0]], err[b], sg[b]).wait()

    def compute(c, b):
        def grp(g, _):
            rid = g * L + iota
            e = (plsc.load_gather(elr[b], [rid, zf])
                 + plsc.load_gather(err[b], [rid, zf]))
            e = jnp.maximum(e, 0.2 * e)
            exv[pl.ds(g * L, L)] = jnp.exp(e)
            return 0
        lax.fori_loop(0, C // L, grp, 0)

        @pl.when(c >= 2)
        def _():
            pltpu.make_async_copy(msg[b], acc.at[didx_all.at[c]],
                                  ss[b]).wait()

        @plsc.parallel_loop(0, C, 1, unroll=4)
        def _edge(e):
            w = plsc.load_gather(exv, [jnp.full((L,), e, jnp.int32)])
            msg[b][e, pl.ds(0, L)] = w * rows[b][e, pl.ds(0, L)]
            msg[b][e, pl.ds(L, L)] = w * rows[b][e, pl.ds(L, L)]
            msg[b][e, pl.ds(2 * L, L)] = w

        pltpu.async_copy(msg[b], acc.at[didx_all.at[c]], ss[b], add=True)

    @plsc.parallel_loop(0, C, 1, unroll=4)
    def _zrow(r):
        for j in range(3):
            msg0[r, pl.ds(j * L, L)] = zv
    for k in range(NPAD // NS // C):
        pltpu.sync_copy(msg0, acc.at[pl.ds((sid * 8 + k) * C, C)])
    plsc.subcore_barrier()

    issue_gathers(0, 0)
    issue_gathers(1, 1)

    def pair(it, _):
        for b in range(2):
            c = 2 * it + b
            wait_gathers(b)
            compute(c, b)

            @pl.when(c < NCHUNKS - 2)
            def _():
                issue_gathers(c + 2, b)
        return 0
    lax.fori_loop(0, (NCHUNKS - 1) // 2, pair, 0)

    wait_gathers(0)
    compute(NCHUNKS - 1, 0)
    pltpu.make_async_copy(msg1, acc.at[didx_all.at[0]], ss1).wait()
    pltpu.make_async_copy(msg0, acc.at[didx_all.at[0]], ss0).wait()

    plsc.subcore_barrier()
    rpw = NPAD // NS
    pltpu.sync_copy(acc.at[pl.ds(sid * rpw, rpw)],
                    out_h.at[cid, pl.ds(sid * rpw, rpw)])


_sc2 = pl.kernel(
    _sc2_body,
    out_type=jax.ShapeDtypeStruct((NC, NPAD, 48), jnp.float32),
    mesh=_MESH,
    compiler_params=_SC_PARAMS,
    scratch_types=[
        pltpu.VMEM_SHARED((NPAD, 48), jnp.float32),
        pltpu.VMEM((NCHUNKS, C), jnp.int32),
        pltpu.VMEM((NCHUNKS, C), jnp.int32),
        pltpu.VMEM((C, 32), jnp.float32),
        pltpu.VMEM((C, 32), jnp.float32),
        pltpu.VMEM((C, 16), jnp.float32),
        pltpu.VMEM((C, 16), jnp.float32),
        pltpu.VMEM((C, 16), jnp.float32),
        pltpu.VMEM((C, 16), jnp.float32),
        pltpu.VMEM((C, 48), jnp.float32),
        pltpu.VMEM((C, 48), jnp.float32),
        pltpu.VMEM((C,), jnp.float32),
        pltpu.SemaphoreType.DMA,
        pltpu.SemaphoreType.DMA,
        pltpu.SemaphoreType.DMA,
        pltpu.SemaphoreType.DMA,
    ],
)


def _sc3_body(zsd_h, zdd_h, src2_h, dst2_h, out_h,
              sidx_all, didx_all, ga0, ga1, gb0, gb1, zout0, zout1,
              sg0, sg1, sw0, sw1):
    cid = lax.axis_index("c")
    sid = lax.axis_index("s")
    wid = cid * NS + sid
    ga = (ga0, ga1)
    gb = (gb0, gb1)
    zout = (zout0, zout1)
    sg = (sg0, sg1)
    sw = (sw0, sw1)
    ebase = wid * EPW

    cbase = wid * NCHUNKS
    pltpu.sync_copy(src2_h.at[pl.ds(cbase, NCHUNKS)], sidx_all)
    pltpu.sync_copy(dst2_h.at[pl.ds(cbase, NCHUNKS)], didx_all)

    def issue_gathers(c, b):
        pltpu.async_copy(zsd_h.at[sidx_all.at[c]], ga[b], sg[b])
        pltpu.async_copy(zdd_h.at[didx_all.at[c]], gb[b], sg[b])

    def wait_gathers(b):
        pltpu.make_async_copy(zsd_h.at[sidx_all.at[0]], ga[b], sg[b]).wait()
        pltpu.make_async_copy(zdd_h.at[didx_all.at[0]], gb[b], sg[b]).wait()

    def compute(c, b):
        base = ebase + c * C

        @pl.when(c >= 2)
        def _():
            pltpu.make_async_copy(zout[b], out_h.at[pl.ds(base, C)],
                                  sw[b]).wait()

        @plsc.parallel_loop(0, C, 1, unroll=4)
        def _edge(e):
            zout[b][e, pl.ds(0, L)] = (ga[b][e, pl.ds(0, L)]
                                       + gb[b][e, pl.ds(0, L)])

        pltpu.async_copy(zout[b], out_h.at[pl.ds(base, C)], sw[b])

    issue_gathers(0, 0)
    issue_gathers(1, 1)

    def pair(it, _):
        for b in range(2):
            c = 2 * it + b
            wait_gathers(b)
            compute(c, b)

            @pl.when(c < NCHUNKS - 2)
            def _():
                issue_gathers(c + 2, b)
        return 0
    lax.fori_loop(0, (NCHUNKS - 1) // 2, pair, 0)

    wait_gathers(0)
    compute(NCHUNKS - 1, 0)
    pltpu.make_async_copy(zout1, out_h.at[pl.ds(ebase, C)], sw1).wait()
    pltpu.make_async_copy(zout0, out_h.at[pl.ds(ebase, C)], sw0).wait()


_sc3 = pl.kernel(
    _sc3_body,
    out_type=jax.ShapeDtypeStruct((E, 16), jnp.float32),
    mesh=_MESH,
    compiler_params=_SC_PARAMS,
    scratch_types=[
        pltpu.VMEM((NCHUNKS, C), jnp.int32),
        pltpu.VMEM((NCHUNKS, C), jnp.int32),
        pltpu.VMEM((C, 16), jnp.float32),
        pltpu.VMEM((C, 16), jnp.float32),
        pltpu.VMEM((C, 16), jnp.float32),
        pltpu.VMEM((C, 16), jnp.float32),
        pltpu.VMEM((C, 16), jnp.float32),
        pltpu.VMEM((C, 16), jnp.float32),
        pltpu.SemaphoreType.DMA,
        pltpu.SemaphoreType.DMA,
        pltpu.SemaphoreType.DMA,
        pltpu.SemaphoreType.DMA,
    ],
)


def kernel(n_feats, edge_index, edge_attr, W1, attn_l1, attn_r1, b1,
           W2, attn_l2, attn_r2, b2, Wp1, bp1, gamma, beta, Wp2, bp2):
    src = edge_index[0].reshape(E // C, C)
    dst = edge_index[1].reshape(E // C, C)
    src1 = edge_index[0].reshape(E // C1, C1)
    dst1 = edge_index[1].reshape(E // C1, C1)
    f32 = jnp.float32

    alT = (jnp.eye(8, dtype=f32)[:, None, :]
           * attn_l1[:, :, None]).reshape(128, 8)
    arT = (jnp.eye(8, dtype=f32)[:, None, :]
           * attn_r1[:, :, None]).reshape(128, 8)
    ex8 = jnp.repeat(jnp.eye(8, dtype=f32), 16, axis=1)
    a2t = jnp.concatenate([attn_l2.reshape(32, 1),
                           attn_r2.reshape(32, 1)], axis=1)
    wa = Wp1[:, :16]
    wsdT = jnp.concatenate([Wp1[:, 16:48].T, Wp1[:, 48:80].T], axis=1)
    waT16 = jnp.concatenate([wa.T, jnp.zeros((16, 8), f32)], axis=1)
    bp1_16 = jnp.concatenate([bp1, jnp.zeros((8,), f32)]).reshape(1, 16)
    g16 = jnp.concatenate([gamma, jnp.ones((8,), f32)]).reshape(1, 16)
    bt16 = jnp.concatenate([beta, jnp.zeros((8,), f32)]).reshape(1, 16)
    w2T = jnp.concatenate([Wp2.T, jnp.zeros((8, 2), f32)], axis=0)
    bp2_r = bp2.reshape(1, 2)

    h, elt, ert = _tc1(n_feats, W1.T, alT, arT)
    out1 = _sc1(h, elt, ert, src1, dst1)
    ht2, elt2, ert2 = _tc2(out1, b1.reshape(1, 128), ex8, W2.T, a2t)

    out2 = _sc2(ht2, elt2, ert2, src, dst)
    zsd, zdd = _tc3(out2, b2.reshape(1, 32), wsdT)

    zp = _sc3(zsd, zdd, src, dst)
    za, ps = _tc4a(edge_attr, zp, waT16, bp1_16)
    return _tc4b(za, ps, g16, bt16, w2T, bp2_r)
